# baseline (speedup 1.0000x reference)
import functools
import numpy as np
import jax
import jax.numpy as jnp
from jax.experimental import pallas as pl
from jax.experimental.pallas import tpu as pltpu

S = 16
V = 8
NB = 10
CUT = 4.0

_RS = 1.0 / np.sqrt(S)
_R3 = 1.0 / np.sqrt(3.0)
_RSV = 1.0 / np.sqrt(S + V)
_RS2V = 1.0 / np.sqrt(S + 2 * V)
_R2 = 1.0 / np.sqrt(2.0)


def _sigmoid(x):
    return 1.0 / (1.0 + jnp.exp(-x))


def _silu(x):
    return x * _sigmoid(x)


def _geom(rel):
    d = jnp.sqrt(jnp.sum(rel * rel, axis=1, keepdims=True))
    dn = jnp.maximum(d, 1e-9)
    sh1 = np.float32(np.sqrt(3.0)) * rel / dn
    return d, sh1


def _radial_h(d, freq, rw1, rb1):
    x = jnp.maximum(d * np.float32(1.0 / CUT), 1e-6)
    xp = x ** 5
    env = 1.0 / x + (-28.0) * xp + 48.0 * xp * x + (-21.0) * xp * x * x
    env = jnp.where(x < 1.0, env, 0.0)
    rbf = jnp.sin(freq * x) * env
    return _silu(jnp.dot(rbf, rw1, preferred_element_type=jnp.float32) + rb1)


def _outer(h, x):
    return jnp.concatenate([h[:, k:k + 1] * x for k in range(S)], axis=1)


def _edge0_body(rel_ref, xg_ref, freq_ref, rw1_ref, rb1_ref, A_ref, Bb_ref,
                out_ref):
    d, sh1 = _geom(rel_ref[...])
    h = _radial_h(d, freq_ref[...], rw1_ref[...], rb1_ref[...])
    xs = xg_ref[...]
    P = _outer(h, xs)
    o32 = (jnp.dot(P, A_ref[...], preferred_element_type=jnp.float32) +
           jnp.dot(xs, Bb_ref[...], preferred_element_type=jnp.float32))
    o32 = o32 * np.float32(_RS)
    scal = o32[:, :S + V]
    vc = o32[:, S + V:S + 2 * V]
    vecs = [vc * sh1[:, m:m + 1] for m in range(3)]
    out_ref[...] = jnp.concatenate([scal] + vecs, axis=1)


def _edge1_body(rel_ref, xg_ref, freq_ref, rw1_ref, rb1_ref, A_ref, Bb_ref,
                Av_ref, Bv_ref, out_ref):
    d, sh1 = _geom(rel_ref[...])
    h = _radial_h(d, freq_ref[...], rw1_ref[...], rb1_ref[...])
    xg = xg_ref[...]
    xs = xg[:, :S]
    xv = [xg[:, S + V * m:S + V * (m + 1)] for m in range(3)]
    dot = (xv[0] * sh1[:, 0:1] + xv[1] * sh1[:, 1:2] +
           xv[2] * sh1[:, 2:3]) * np.float32(_R3)
    u = jnp.concatenate([xs, dot], axis=1)
    Q = _outer(h, u)
    o32 = (jnp.dot(Q, A_ref[...], preferred_element_type=jnp.float32) +
           jnp.dot(u, Bb_ref[...], preferred_element_type=jnp.float32))
    scal = o32[:, :S + V] * np.float32(_RSV)
    cSV = o32[:, S + V:S + 2 * V]
    crs = [
        (xv[1] * sh1[:, 2:3] - xv[2] * sh1[:, 1:2]) * np.float32(_R2),
        (xv[2] * sh1[:, 0:1] - xv[0] * sh1[:, 2:3]) * np.float32(_R2),
        (xv[0] * sh1[:, 1:2] - xv[1] * sh1[:, 0:1]) * np.float32(_R2),
    ]
    Av = Av_ref[...]
    Bv = Bv_ref[...]
    vecs = []
    for m in range(3):
        G = jnp.concatenate([xv[m], crs[m]], axis=1)
        R = _outer(h, G)
        v0c = (jnp.dot(R, Av, preferred_element_type=jnp.float32) +
               jnp.dot(G, Bv, preferred_element_type=jnp.float32))
        vecs.append((cSV * sh1[:, m:m + 1] + v0c) * np.float32(_RS2V))
    out_ref[...] = jnp.concatenate([scal] + vecs, axis=1)


def _edge2_body(rel_ref, xg_ref, freq_ref, rw1_ref, rb1_ref, A_ref, Bb_ref,
                out_ref):
    d, sh1 = _geom(rel_ref[...])
    h = _radial_h(d, freq_ref[...], rw1_ref[...], rb1_ref[...])
    xg = xg_ref[...]
    xs = xg[:, :S]
    xv = [xg[:, S + V * m:S + V * (m + 1)] for m in range(3)]
    dot = (xv[0] * sh1[:, 0:1] + xv[1] * sh1[:, 1:2] +
           xv[2] * sh1[:, 2:3]) * np.float32(_R3)
    u = jnp.concatenate([xs, dot], axis=1)
    Q = _outer(h, u)
    out_ref[...] = (jnp.dot(Q, A_ref[...], preferred_element_type=jnp.float32)
                    + jnp.dot(u, Bb_ref[...],
                              preferred_element_type=jnp.float32)
                    ) * np.float32(_RSV)


def _run_edge(body, n_extra, rel, xg, consts, out_dim, eb):
    E = rel.shape[0]
    grid = E // eb
    full = lambda a: pl.BlockSpec(a.shape, lambda i: (0,) * a.ndim)
    in_specs = [
        pl.BlockSpec((eb, rel.shape[1]), lambda i: (i, 0)),
        pl.BlockSpec((eb, xg.shape[1]), lambda i: (i, 0)),
    ] + [full(c) for c in consts]
    return pl.pallas_call(
        body,
        grid=(grid,),
        in_specs=in_specs,
        out_specs=pl.BlockSpec((eb, out_dim), lambda i: (i, 0)),
        out_shape=jax.ShapeDtypeStruct((E, out_dim), jnp.float32),
    )(rel, xg, *consts)


def _embed_body(z_ref, embed_ref, out_ref):
    z = z_ref[...]
    emb = embed_ref[...]
    acc = jnp.zeros((z.shape[0], S), jnp.float32)
    for c in range(emb.shape[0]):
        acc = acc + jnp.where(z == c, 1.0, 0.0) * emb[c][None, :]
    out_ref[...] = acc


def _gate_body(h_ref, out_ref):
    h = h_ref[...]
    scal = _silu(h[:, :S])
    g = _sigmoid(h[:, S:S + V])
    vecs = [h[:, S + V + V * m:S + V + V * (m + 1)] * g for m in range(3)]
    out_ref[...] = jnp.concatenate([scal] + vecs, axis=1)


def _readout_body(h_ref, b_ref, dw1_ref, db1_ref, dw2_ref, db2_ref, out_ref,
                  acc_ref, *, ng, nblocks):
    i = pl.program_id(0)

    @pl.when(i == 0)
    def _init():
        acc_ref[...] = jnp.zeros_like(acc_ref)

    h = h_ref[...]
    t = jax.nn.relu(jnp.dot(h, dw1_ref[...],
                            preferred_element_type=jnp.float32) + db1_ref[...])
    y = jnp.dot(t, dw2_ref[...], preferred_element_type=jnp.float32) \
        + db2_ref[...]
    b = b_ref[...]
    gid = jax.lax.broadcasted_iota(jnp.int32, (1, ng), 1)
    onehot = jnp.where(b == gid, 1.0, 0.0)
    sums = jnp.sum(onehot * y, axis=0, keepdims=True)
    cnts = jnp.sum(onehot, axis=0, keepdims=True)
    acc_ref[0:1, :] += sums
    acc_ref[1:2, :] += cnts

    @pl.when(i == nblocks - 1)
    def _fin():
        out_ref[...] = acc_ref[0:1, :] / jnp.maximum(acc_ref[1:2, :], 1.0)


def kernel(pos, z, edge_index, batch, embed,
           freq0, rw1_0, rb1_0, rw2_0, rb2_0,
           freq1, rw1_1, rb1_1, rw2_1, rb2_1,
           freq2, rw1_2, rb1_2, rw2_2, rb2_2,
           dw1, db1, dw2, db2):
    N = pos.shape[0]
    E = edge_index.shape[1]
    NG = 16
    MAXZ = embed.shape[0]

    src = edge_index[0].astype(jnp.int32)
    tgt = edge_index[1].astype(jnp.int32)

    eb = 1280 if E % 1280 == 0 else E
    nb = 1000 if N % 1000 == 0 else N

    A0 = jnp.concatenate([
        rw2_0[:, :S * (S + V)].reshape(S, S, S + V).reshape(S * S, S + V),
        rw2_0[:, S * (S + V):].reshape(S, S, V).reshape(S * S, V)], axis=1)
    B0 = jnp.concatenate([
        rb2_0[:S * (S + V)].reshape(S, S + V),
        rb2_0[S * (S + V):].reshape(S, V)], axis=1)

    A1s = rw2_1[:, :576].reshape(S, S + V, S + V).reshape(S * (S + V), S + V)
    A1sv = jnp.zeros((S, S + V, V), jnp.float32).at[:, :S, :].set(
        rw2_1[:, 576:704].reshape(S, S, V)).reshape(S * (S + V), V)
    A1 = jnp.concatenate([A1s, A1sv], axis=1)
    B1s = rb2_1[:576].reshape(S + V, S + V)
    B1sv = jnp.zeros((S + V, V), jnp.float32).at[:S, :].set(
        rb2_1[576:704].reshape(S, V))
    B1 = jnp.concatenate([B1s, B1sv], axis=1)
    A1v = rw2_1[:, 704:832].reshape(S, 2 * V, V).reshape(S * 2 * V, V)
    B1v = rb2_1[704:832].reshape(2 * V, V)

    A2 = rw2_2.reshape(S, S + V, S).reshape(S * (S + V), S)
    B2 = rb2_2.reshape(S + V, S)

    f0 = freq0.reshape(1, NB)
    f1 = freq1.reshape(1, NB)
    f2 = freq2.reshape(1, NB)
    rb1_0r = rb1_0.reshape(1, S)
    rb1_1r = rb1_1.reshape(1, S)
    rb1_2r = rb1_2.reshape(1, S)

    rel = _gather_rows(pos, tgt) - _gather_rows(pos, src)

    x0 = pl.pallas_call(
        _embed_body,
        grid=(N // nb,),
        in_specs=[pl.BlockSpec((nb, 1), lambda i: (i, 0)),
                  pl.BlockSpec((MAXZ, S), lambda i: (0, 0))],
        out_specs=pl.BlockSpec((nb, S), lambda i: (i, 0)),
        out_shape=jax.ShapeDtypeStruct((N, S), jnp.float32),
    )(z.astype(jnp.int32).reshape(N, 1), embed)

    xg = _gather_rows(x0, src)
    msg = _run_edge(_edge0_body, 5, rel, xg,
                    [f0, rw1_0, rb1_0r, A0, B0], 48, eb)
    h = _segsum(msg, tgt, N)
    x = pl.pallas_call(
        _gate_body,
        grid=(N // nb,),
        in_specs=[pl.BlockSpec((nb, 48), lambda i: (i, 0))],
        out_specs=pl.BlockSpec((nb, 40), lambda i: (i, 0)),
        out_shape=jax.ShapeDtypeStruct((N, 40), jnp.float32),
    )(h)

    xg = _gather_rows(x, src)
    msg = _run_edge(_edge1_body, 7, rel, xg,
                    [f1, rw1_1, rb1_1r, A1, B1, A1v, B1v], 48, eb)
    h = _segsum(msg, tgt, N)
    x = pl.pallas_call(
        _gate_body,
        grid=(N // nb,),
        in_specs=[pl.BlockSpec((nb, 48), lambda i: (i, 0))],
        out_specs=pl.BlockSpec((nb, 40), lambda i: (i, 0)),
        out_shape=jax.ShapeDtypeStruct((N, 40), jnp.float32),
    )(h)

    xg = _gather_rows(x, src)
    msg = _run_edge(_edge2_body, 5, rel, xg,
                    [f2, rw1_2, rb1_2r, A2, B2], 16, eb)
    h = _segsum(msg, tgt, N)

    nblocks = N // nb
    out = pl.pallas_call(
        functools.partial(_readout_body, ng=NG, nblocks=nblocks),
        grid=(nblocks,),
        in_specs=[pl.BlockSpec((nb, S), lambda i: (i, 0)),
                  pl.BlockSpec((nb, 1), lambda i: (i, 0)),
                  pl.BlockSpec((S, 2 * S), lambda i: (0, 0)),
                  pl.BlockSpec((1, 2 * S), lambda i: (0, 0)),
                  pl.BlockSpec((2 * S, 1), lambda i: (0, 0)),
                  pl.BlockSpec((1, 1), lambda i: (0, 0))],
        out_specs=pl.BlockSpec((1, NG), lambda i: (0, 0)),
        out_shape=jax.ShapeDtypeStruct((1, NG), jnp.float32),
        scratch_shapes=[pltpu.VMEM((2, NG), jnp.float32)],
    )(h, batch.astype(jnp.int32).reshape(N, 1), dw1,
      db1.reshape(1, 2 * S), dw2, db2.reshape(1, 1))
    return out[0]


def _gather_rows(table, idx):
    return table[idx]


def _segsum(vals, idx, n):
    return jax.ops.segment_sum(vals, idx, num_segments=n)


# speedup vs baseline: 1.1034x; 1.1034x over previous
import functools
import numpy as np
import jax
import jax.numpy as jnp
from jax import lax
from jax.experimental import pallas as pl
from jax.experimental.pallas import tpu as pltpu
from jax.experimental.pallas import tpu_sc as plsc

S = 16
V = 8
NB = 10
CUT = 4.0

_RS = 1.0 / np.sqrt(S)
_R3 = 1.0 / np.sqrt(3.0)
_RSV = 1.0 / np.sqrt(S + V)
_RS2V = 1.0 / np.sqrt(S + 2 * V)
_R2 = 1.0 / np.sqrt(2.0)


def _sigmoid(x):
    return 1.0 / (1.0 + jnp.exp(-x))


def _silu(x):
    return x * _sigmoid(x)


def _geom(rel):
    d = jnp.sqrt(jnp.sum(rel * rel, axis=1, keepdims=True))
    dn = jnp.maximum(d, 1e-9)
    sh1 = np.float32(np.sqrt(3.0)) * rel / dn
    return d, sh1


def _radial_h(d, freq, rw1, rb1):
    x = jnp.maximum(d * np.float32(1.0 / CUT), 1e-6)
    xp = x ** 5
    env = 1.0 / x + (-28.0) * xp + 48.0 * xp * x + (-21.0) * xp * x * x
    env = jnp.where(x < 1.0, env, 0.0)
    rbf = jnp.sin(freq * x) * env
    return _silu(jnp.dot(rbf, rw1, preferred_element_type=jnp.float32) + rb1)


def _outer(h, x):
    return jnp.concatenate([h[:, k:k + 1] * x for k in range(S)], axis=1)


def _edge0_body(rel_ref, xg_ref, freq_ref, rw1_ref, rb1_ref, A_ref, Bb_ref,
                out_ref):
    d, sh1 = _geom(rel_ref[:, :3])
    h = _radial_h(d, freq_ref[...], rw1_ref[...], rb1_ref[...])
    xs = xg_ref[...]
    P = _outer(h, xs)
    o32 = (jnp.dot(P, A_ref[...], preferred_element_type=jnp.float32) +
           jnp.dot(xs, Bb_ref[...], preferred_element_type=jnp.float32))
    o32 = o32 * np.float32(_RS)
    scal = o32[:, :S + V]
    vc = o32[:, S + V:S + 2 * V]
    vecs = [vc * sh1[:, m:m + 1] for m in range(3)]
    out_ref[...] = jnp.concatenate([scal] + vecs, axis=1)


def _edge1_body(rel_ref, xg_ref, freq_ref, rw1_ref, rb1_ref, A_ref, Bb_ref,
                Av_ref, Bv_ref, out_ref):
    d, sh1 = _geom(rel_ref[:, :3])
    h = _radial_h(d, freq_ref[...], rw1_ref[...], rb1_ref[...])
    xg = xg_ref[...]
    xs = xg[:, :S]
    xv = [xg[:, S + V * m:S + V * (m + 1)] for m in range(3)]
    dot = (xv[0] * sh1[:, 0:1] + xv[1] * sh1[:, 1:2] +
           xv[2] * sh1[:, 2:3]) * np.float32(_R3)
    u = jnp.concatenate([xs, dot], axis=1)
    Q = _outer(h, u)
    o32 = (jnp.dot(Q, A_ref[...], preferred_element_type=jnp.float32) +
           jnp.dot(u, Bb_ref[...], preferred_element_type=jnp.float32))
    scal = o32[:, :S + V] * np.float32(_RSV)
    cSV = o32[:, S + V:S + 2 * V]
    crs = [
        (xv[1] * sh1[:, 2:3] - xv[2] * sh1[:, 1:2]) * np.float32(_R2),
        (xv[2] * sh1[:, 0:1] - xv[0] * sh1[:, 2:3]) * np.float32(_R2),
        (xv[0] * sh1[:, 1:2] - xv[1] * sh1[:, 0:1]) * np.float32(_R2),
    ]
    Av = Av_ref[...]
    Bv = Bv_ref[...]
    vecs = []
    for m in range(3):
        G = jnp.concatenate([xv[m], crs[m]], axis=1)
        R = _outer(h, G)
        v0c = (jnp.dot(R, Av, preferred_element_type=jnp.float32) +
               jnp.dot(G, Bv, preferred_element_type=jnp.float32))
        vecs.append((cSV * sh1[:, m:m + 1] + v0c) * np.float32(_RS2V))
    out_ref[...] = jnp.concatenate([scal] + vecs, axis=1)


def _edge2_body(rel_ref, xg_ref, freq_ref, rw1_ref, rb1_ref, A_ref, Bb_ref,
                out_ref):
    d, sh1 = _geom(rel_ref[:, :3])
    h = _radial_h(d, freq_ref[...], rw1_ref[...], rb1_ref[...])
    xg = xg_ref[...]
    xs = xg[:, :S]
    xv = [xg[:, S + V * m:S + V * (m + 1)] for m in range(3)]
    dot = (xv[0] * sh1[:, 0:1] + xv[1] * sh1[:, 1:2] +
           xv[2] * sh1[:, 2:3]) * np.float32(_R3)
    u = jnp.concatenate([xs, dot], axis=1)
    Q = _outer(h, u)
    out_ref[...] = (jnp.dot(Q, A_ref[...], preferred_element_type=jnp.float32)
                    + jnp.dot(u, Bb_ref[...],
                              preferred_element_type=jnp.float32)
                    ) * np.float32(_RSV)


def _run_edge(body, n_extra, rel, xg, consts, out_dim, eb):
    E = rel.shape[0]
    grid = E // eb
    full = lambda a: pl.BlockSpec(a.shape, lambda i: (0,) * a.ndim)
    in_specs = [
        pl.BlockSpec((eb, rel.shape[1]), lambda i: (i, 0)),
        pl.BlockSpec((eb, xg.shape[1]), lambda i: (i, 0)),
    ] + [full(c) for c in consts]
    return pl.pallas_call(
        body,
        grid=(grid,),
        in_specs=in_specs,
        out_specs=pl.BlockSpec((eb, out_dim), lambda i: (i, 0)),
        out_shape=jax.ShapeDtypeStruct((E, out_dim), jnp.float32),
    )(rel, xg, *consts)


def _embed_body(z_ref, embed_ref, out_ref):
    z = z_ref[...]
    emb = embed_ref[...]
    acc = jnp.zeros((z.shape[0], S), jnp.float32)
    for c in range(emb.shape[0]):
        acc = acc + jnp.where(z == c, 1.0, 0.0) * emb[c][None, :]
    out_ref[...] = acc


def _gate_body(ha_ref, hb_ref, out_ref):
    h = ha_ref[...] + hb_ref[...]
    scal = _silu(h[:, :S])
    g = _sigmoid(h[:, S:S + V])
    vecs = [h[:, S + V + V * m:S + V + V * (m + 1)] * g for m in range(3)]
    out_ref[...] = jnp.concatenate([scal] + vecs, axis=1)


def _readout_body(ha_ref, hb_ref, b_ref, dw1_ref, db1_ref, dw2_ref, db2_ref,
                  out_ref, acc_ref, *, ng, nblocks):
    i = pl.program_id(0)

    @pl.when(i == 0)
    def _init():
        acc_ref[...] = jnp.zeros_like(acc_ref)

    h = ha_ref[...] + hb_ref[...]
    t = jax.nn.relu(jnp.dot(h, dw1_ref[...],
                            preferred_element_type=jnp.float32) + db1_ref[...])
    y = jnp.dot(t, dw2_ref[...], preferred_element_type=jnp.float32) \
        + db2_ref[...]
    b = b_ref[...]
    gid = jax.lax.broadcasted_iota(jnp.int32, (1, ng), 1)
    onehot = jnp.where(b == gid, 1.0, 0.0)
    sums = jnp.sum(onehot * y, axis=0, keepdims=True)
    cnts = jnp.sum(onehot, axis=0, keepdims=True)
    acc_ref[0:1, :] += sums
    acc_ref[1:2, :] += cnts

    @pl.when(i == nblocks - 1)
    def _fin():
        out_ref[...] = acc_ref[0:1, :] / jnp.maximum(acc_ref[1:2, :], 1.0)


_NW = 32
_CW = 128


def _sc_mesh():
    return plsc.VectorSubcoreMesh(core_axis_name="c", subcore_axis_name="s")


def _wid():
    return lax.axis_index("s") * 2 + lax.axis_index("c")


def _geom_sc(n, ept):
    ch = ept // _CW

    def body(pos_hbm, src_hbm, tgt_hbm, rel_hbm, pos_v, si_v, ti_v, rel_v):
        w = _wid()
        pltpu.sync_copy(pos_hbm, pos_v)
        pltpu.sync_copy(src_hbm.at[w], si_v)
        pltpu.sync_copy(tgt_hbm.at[w], ti_v)
        iota = lax.broadcasted_iota(jnp.int32, (16,), 0)

        def step(i, carry):
            j = i // 8
            k = i % 8
            sl = pl.ds(k * 16, 16)
            si = si_v[j, sl] * 3
            ti = ti_v[j, sl] * 3
            row = (i * 16 + iota) * 4
            for c in range(3):
                gs = plsc.load_gather(pos_v, [si + c])
                gt = plsc.load_gather(pos_v, [ti + c])
                plsc.store_scatter(rel_v, [row + c], gt - gs)
            return carry

        lax.fori_loop(0, ch * 8, step, 0)
        pltpu.sync_copy(rel_v, rel_hbm.at[pl.ds(w * ept * 4, ept * 4)])

    return pl.kernel(
        body,
        mesh=_sc_mesh(),
        compiler_params=pltpu.CompilerParams(needs_layout_passes=False),
        out_type=jax.ShapeDtypeStruct((_NW * ept * 4,), jnp.float32),
        scratch_types=[
            pltpu.VMEM((n * 3,), jnp.float32),
            pltpu.VMEM((ch, _CW), jnp.int32),
            pltpu.VMEM((ch, _CW), jnp.int32),
            pltpu.VMEM((ept * 4,), jnp.float32),
        ],
    )


def _scat_sc(n, ept, f):
    ch = ept // _CW
    nrow = n + (-n % 128) + 128
    rpt = nrow // 16

    def body(msg_hbm, tgt_hbm, zero_hbm, out_hbm, idx_v, buf_v, acc_sh):
        c = lax.axis_index("c")
        s = lax.axis_index("s")
        w = s * 2 + c
        pltpu.sync_copy(tgt_hbm.at[w], idx_v)
        pltpu.sync_copy(zero_hbm.at[pl.ds(s * rpt, rpt)],
                        acc_sh.at[pl.ds(s * rpt, rpt)])
        plsc.subcore_barrier()

        def step(j, carry):
            pltpu.sync_copy(msg_hbm.at[pl.ds(w * ept + j * _CW, _CW)], buf_v)
            for k in range(_CW // 16):
                idx = idx_v[j, pl.ds(k * 16, 16)]
                pltpu.sync_copy(buf_v.at[pl.ds(k * 16, 16)],
                                acc_sh.at[idx], add=True)
            return carry

        lax.fori_loop(0, ch, step, 0)
        plsc.subcore_barrier()
        pltpu.sync_copy(acc_sh.at[pl.ds(s * rpt, rpt)],
                        out_hbm.at[c, pl.ds(s * rpt, rpt)])

    return pl.kernel(
        body,
        mesh=_sc_mesh(),
        compiler_params=pltpu.CompilerParams(needs_layout_passes=False,
                                             use_tc_tiling_on_sc=False),
        out_type=jax.ShapeDtypeStruct((2, nrow, f), jnp.float32),
        scratch_types=[
            pltpu.VMEM((ch, _CW), jnp.int32),
            pltpu.VMEM((_CW, f), jnp.float32),
            pltpu.VMEM_SHARED((nrow, f), jnp.float32),
        ],
    )


def kernel(pos, z, edge_index, batch, embed,
           freq0, rw1_0, rb1_0, rw2_0, rb2_0,
           freq1, rw1_1, rb1_1, rw2_1, rb2_1,
           freq2, rw1_2, rb1_2, rw2_2, rb2_2,
           dw1, db1, dw2, db2):
    N = pos.shape[0]
    E = edge_index.shape[1]
    NG = 16
    MAXZ = embed.shape[0]

    src = edge_index[0].astype(jnp.int32)
    tgt = edge_index[1].astype(jnp.int32)

    ep = -E % (_NW * _CW)
    E_pad = E + ep
    ept = E_pad // _NW
    src_p = jnp.concatenate([src, jnp.zeros((ep,), jnp.int32)])
    tgt_p = jnp.concatenate([tgt, jnp.full((ep,), N, jnp.int32)])
    src3 = src_p.reshape(_NW, ept // _CW, _CW)
    tgt3 = tgt_p.reshape(_NW, ept // _CW, _CW)

    eb = 1280 if E_pad % 1280 == 0 else E_pad
    nb = 1000 if N % 1000 == 0 else N

    A0 = jnp.concatenate([
        rw2_0[:, :S * (S + V)].reshape(S, S, S + V).reshape(S * S, S + V),
        rw2_0[:, S * (S + V):].reshape(S, S, V).reshape(S * S, V)], axis=1)
    B0 = jnp.concatenate([
        rb2_0[:S * (S + V)].reshape(S, S + V),
        rb2_0[S * (S + V):].reshape(S, V)], axis=1)

    A1s = rw2_1[:, :576].reshape(S, S + V, S + V).reshape(S * (S + V), S + V)
    A1sv = jnp.zeros((S, S + V, V), jnp.float32).at[:, :S, :].set(
        rw2_1[:, 576:704].reshape(S, S, V)).reshape(S * (S + V), V)
    A1 = jnp.concatenate([A1s, A1sv], axis=1)
    B1s = rb2_1[:576].reshape(S + V, S + V)
    B1sv = jnp.zeros((S + V, V), jnp.float32).at[:S, :].set(
        rb2_1[576:704].reshape(S, V))
    B1 = jnp.concatenate([B1s, B1sv], axis=1)
    A1v = rw2_1[:, 704:832].reshape(S, 2 * V, V).reshape(S * 2 * V, V)
    B1v = rb2_1[704:832].reshape(2 * V, V)

    A2 = rw2_2.reshape(S, S + V, S).reshape(S * (S + V), S)
    B2 = rb2_2.reshape(S + V, S)

    f0 = freq0.reshape(1, NB)
    f1 = freq1.reshape(1, NB)
    f2 = freq2.reshape(1, NB)
    rb1_0r = rb1_0.reshape(1, S)
    rb1_1r = rb1_1.reshape(1, S)
    rb1_2r = rb1_2.reshape(1, S)

    rel = _geom_sc(N, ept)(pos.reshape(-1), src3, tgt3).reshape(E_pad, 4)

    x0 = pl.pallas_call(
        _embed_body,
        grid=(N // nb,),
        in_specs=[pl.BlockSpec((nb, 1), lambda i: (i, 0)),
                  pl.BlockSpec((MAXZ, S), lambda i: (0, 0))],
        out_specs=pl.BlockSpec((nb, S), lambda i: (i, 0)),
        out_shape=jax.ShapeDtypeStruct((N, S), jnp.float32),
    )(z.astype(jnp.int32).reshape(N, 1), embed)

    nrow = N + (-N % 128) + 128
    zeros48 = jnp.zeros((nrow, 48), jnp.float32)
    zeros16 = jnp.zeros((nrow, 16), jnp.float32)

    def _gate(hp):
        return pl.pallas_call(
            _gate_body,
            grid=(N // nb,),
            in_specs=[pl.BlockSpec((nb, 48), lambda i: (i, 0)),
                      pl.BlockSpec((nb, 48), lambda i: (i, 0))],
            out_specs=pl.BlockSpec((nb, 40), lambda i: (i, 0)),
            out_shape=jax.ShapeDtypeStruct((N, 40), jnp.float32),
        )(hp[0, :N], hp[1, :N])

    xg = _gather_rows(x0, src_p)
    msg = _run_edge(_edge0_body, 5, rel, xg,
                    [f0, rw1_0, rb1_0r, A0, B0], 48, eb)
    x = _gate(_scat_sc(N, ept, 48)(msg, tgt3, zeros48))

    xg = _gather_rows(x, src_p)
    msg = _run_edge(_edge1_body, 7, rel, xg,
                    [f1, rw1_1, rb1_1r, A1, B1, A1v, B1v], 48, eb)
    x = _gate(_scat_sc(N, ept, 48)(msg, tgt3, zeros48))

    xg = _gather_rows(x, src_p)
    msg = _run_edge(_edge2_body, 5, rel, xg,
                    [f2, rw1_2, rb1_2r, A2, B2], 16, eb)
    hp = _scat_sc(N, ept, 16)(msg, tgt3, zeros16)

    nblocks = N // nb
    out = pl.pallas_call(
        functools.partial(_readout_body, ng=NG, nblocks=nblocks),
        grid=(nblocks,),
        in_specs=[pl.BlockSpec((nb, S), lambda i: (i, 0)),
                  pl.BlockSpec((nb, S), lambda i: (i, 0)),
                  pl.BlockSpec((nb, 1), lambda i: (i, 0)),
                  pl.BlockSpec((S, 2 * S), lambda i: (0, 0)),
                  pl.BlockSpec((1, 2 * S), lambda i: (0, 0)),
                  pl.BlockSpec((2 * S, 1), lambda i: (0, 0)),
                  pl.BlockSpec((1, 1), lambda i: (0, 0))],
        out_specs=pl.BlockSpec((1, NG), lambda i: (0, 0)),
        out_shape=jax.ShapeDtypeStruct((1, NG), jnp.float32),
        scratch_shapes=[pltpu.VMEM((2, NG), jnp.float32)],
    )(hp[0, :N], hp[1, :N], batch.astype(jnp.int32).reshape(N, 1), dw1,
      db1.reshape(1, 2 * S), dw2, db2.reshape(1, 1))
    return out[0]


def _gather_rows(table, idx):
    return table[idx]


def _segsum(vals, idx, n):
    return jax.ops.segment_sum(vals, idx, num_segments=n + 1)[:n]


# speedup vs baseline: 1.3379x; 1.2125x over previous
import functools
import numpy as np
import jax
import jax.numpy as jnp
from jax import lax
from jax.experimental import pallas as pl
from jax.experimental.pallas import tpu as pltpu
from jax.experimental.pallas import tpu_sc as plsc

S = 16
V = 8
NB = 10
CUT = 4.0

_RS = 1.0 / np.sqrt(S)
_R3 = 1.0 / np.sqrt(3.0)
_RSV = 1.0 / np.sqrt(S + V)
_RS2V = 1.0 / np.sqrt(S + 2 * V)
_R2 = 1.0 / np.sqrt(2.0)


def _sigmoid(x):
    return 1.0 / (1.0 + jnp.exp(-x))


def _silu(x):
    return x * _sigmoid(x)


def _geom(rel):
    d = jnp.sqrt(jnp.sum(rel * rel, axis=1, keepdims=True))
    dn = jnp.maximum(d, 1e-9)
    sh1 = np.float32(np.sqrt(3.0)) * rel / dn
    return d, sh1


def _radial_h(d, freq, rw1, rb1):
    x = jnp.maximum(d * np.float32(1.0 / CUT), 1e-6)
    xp = x ** 5
    env = 1.0 / x + (-28.0) * xp + 48.0 * xp * x + (-21.0) * xp * x * x
    env = jnp.where(x < 1.0, env, 0.0)
    rbf = jnp.sin(freq * x) * env
    return _silu(jnp.dot(rbf, rw1, preferred_element_type=jnp.float32) + rb1)


def _outer(h, x):
    return jnp.concatenate([h[:, k:k + 1] * x for k in range(S)], axis=1)


def _edge0_body(rel_ref, xg_ref, freq_ref, rw1_ref, rb1_ref, A_ref, Bb_ref,
                out_ref):
    d, sh1 = _geom(rel_ref[:, :3])
    h = _radial_h(d, freq_ref[...], rw1_ref[...], rb1_ref[...])
    xs = xg_ref[...]
    P = _outer(h, xs)
    o32 = (jnp.dot(P, A_ref[...], preferred_element_type=jnp.float32) +
           jnp.dot(xs, Bb_ref[...], preferred_element_type=jnp.float32))
    o32 = o32 * np.float32(_RS)
    scal = o32[:, :S + V]
    vc = o32[:, S + V:S + 2 * V]
    vecs = [vc * sh1[:, m:m + 1] for m in range(3)]
    out_ref[...] = jnp.concatenate([scal] + vecs, axis=1)


def _edge1_body(rel_ref, xg_ref, freq_ref, rw1_ref, rb1_ref, A_ref, Bb_ref,
                Av_ref, Bv_ref, out_ref):
    d, sh1 = _geom(rel_ref[:, :3])
    h = _radial_h(d, freq_ref[...], rw1_ref[...], rb1_ref[...])
    xg = xg_ref[...]
    xs = xg[:, :S]
    xv = [xg[:, S + V * m:S + V * (m + 1)] for m in range(3)]
    dot = (xv[0] * sh1[:, 0:1] + xv[1] * sh1[:, 1:2] +
           xv[2] * sh1[:, 2:3]) * np.float32(_R3)
    u = jnp.concatenate([xs, dot], axis=1)
    Q = _outer(h, u)
    o32 = (jnp.dot(Q, A_ref[...], preferred_element_type=jnp.float32) +
           jnp.dot(u, Bb_ref[...], preferred_element_type=jnp.float32))
    scal = o32[:, :S + V] * np.float32(_RSV)
    cSV = o32[:, S + V:S + 2 * V]
    crs = [
        (xv[1] * sh1[:, 2:3] - xv[2] * sh1[:, 1:2]) * np.float32(_R2),
        (xv[2] * sh1[:, 0:1] - xv[0] * sh1[:, 2:3]) * np.float32(_R2),
        (xv[0] * sh1[:, 1:2] - xv[1] * sh1[:, 0:1]) * np.float32(_R2),
    ]
    Av = Av_ref[...]
    Bv = Bv_ref[...]
    vecs = []
    for m in range(3):
        G = jnp.concatenate([xv[m], crs[m]], axis=1)
        R = _outer(h, G)
        v0c = (jnp.dot(R, Av, preferred_element_type=jnp.float32) +
               jnp.dot(G, Bv, preferred_element_type=jnp.float32))
        vecs.append((cSV * sh1[:, m:m + 1] + v0c) * np.float32(_RS2V))
    out_ref[...] = jnp.concatenate([scal] + vecs, axis=1)


def _edge2_body(rel_ref, xg_ref, freq_ref, rw1_ref, rb1_ref, A_ref, Bb_ref,
                out_ref):
    d, sh1 = _geom(rel_ref[:, :3])
    h = _radial_h(d, freq_ref[...], rw1_ref[...], rb1_ref[...])
    xg = xg_ref[...]
    xs = xg[:, :S]
    xv = [xg[:, S + V * m:S + V * (m + 1)] for m in range(3)]
    dot = (xv[0] * sh1[:, 0:1] + xv[1] * sh1[:, 1:2] +
           xv[2] * sh1[:, 2:3]) * np.float32(_R3)
    u = jnp.concatenate([xs, dot], axis=1)
    Q = _outer(h, u)
    out_ref[...] = (jnp.dot(Q, A_ref[...], preferred_element_type=jnp.float32)
                    + jnp.dot(u, Bb_ref[...],
                              preferred_element_type=jnp.float32)
                    ) * np.float32(_RSV)


def _run_edge(body, n_extra, rel, xg, consts, out_dim, eb):
    E = rel.shape[0]
    grid = E // eb
    full = lambda a: pl.BlockSpec(a.shape, lambda i: (0,) * a.ndim)
    in_specs = [
        pl.BlockSpec((eb, rel.shape[1]), lambda i: (i, 0)),
        pl.BlockSpec((eb, xg.shape[1]), lambda i: (i, 0)),
    ] + [full(c) for c in consts]
    return pl.pallas_call(
        body,
        grid=(grid,),
        in_specs=in_specs,
        out_specs=pl.BlockSpec((eb, out_dim), lambda i: (i, 0)),
        out_shape=jax.ShapeDtypeStruct((E, out_dim), jnp.float32),
    )(rel, xg, *consts)


def _embed_body(z_ref, embed_ref, out_ref):
    z = z_ref[...]
    emb = embed_ref[...]
    acc = jnp.zeros((z.shape[0], S), jnp.float32)
    for c in range(emb.shape[0]):
        acc = acc + jnp.where(z == c, 1.0, 0.0) * emb[c][None, :]
    out_ref[...] = acc


def _gate_body(ha_ref, hb_ref, out_ref):
    h = ha_ref[...] + hb_ref[...]
    scal = _silu(h[:, :S])
    g = _sigmoid(h[:, S:S + V])
    vecs = [h[:, S + V + V * m:S + V + V * (m + 1)] * g for m in range(3)]
    pad = jnp.zeros((h.shape[0], V), h.dtype)
    out_ref[...] = jnp.concatenate([scal] + vecs + [pad], axis=1)


def _readout_body(ha_ref, hb_ref, b_ref, dw1_ref, db1_ref, dw2_ref, db2_ref,
                  out_ref, acc_ref, *, ng, nblocks):
    i = pl.program_id(0)

    @pl.when(i == 0)
    def _init():
        acc_ref[...] = jnp.zeros_like(acc_ref)

    h = ha_ref[...] + hb_ref[...]
    t = jax.nn.relu(jnp.dot(h, dw1_ref[...],
                            preferred_element_type=jnp.float32) + db1_ref[...])
    y = jnp.dot(t, dw2_ref[...], preferred_element_type=jnp.float32) \
        + db2_ref[...]
    b = b_ref[...]
    gid = jax.lax.broadcasted_iota(jnp.int32, (1, ng), 1)
    onehot = jnp.where(b == gid, 1.0, 0.0)
    sums = jnp.sum(onehot * y, axis=0, keepdims=True)
    cnts = jnp.sum(onehot, axis=0, keepdims=True)
    acc_ref[0:1, :] += sums
    acc_ref[1:2, :] += cnts

    @pl.when(i == nblocks - 1)
    def _fin():
        out_ref[...] = acc_ref[0:1, :] / jnp.maximum(acc_ref[1:2, :], 1.0)


_NW = 32
_CW = 128


def _sc_mesh():
    return plsc.VectorSubcoreMesh(core_axis_name="c", subcore_axis_name="s")


def _wid():
    return lax.axis_index("s") * 2 + lax.axis_index("c")


def _geom_sc(n, ept):
    ch = ept // _CW

    def body(pos_hbm, src_hbm, tgt_hbm, rel_hbm, pos_v, si_v, ti_v, rel_v):
        w = _wid()
        pltpu.sync_copy(pos_hbm, pos_v)
        pltpu.sync_copy(src_hbm.at[w], si_v)
        pltpu.sync_copy(tgt_hbm.at[w], ti_v)
        iota = lax.broadcasted_iota(jnp.int32, (16,), 0)

        def step(i, carry):
            j = i // 8
            k = i % 8
            sl = pl.ds(k * 16, 16)
            si = si_v[j, sl] * 3
            ti = ti_v[j, sl] * 3
            row = (i * 16 + iota) * 4
            for c in range(3):
                gs = plsc.load_gather(pos_v, [si + c])
                gt = plsc.load_gather(pos_v, [ti + c])
                plsc.store_scatter(rel_v, [row + c], gt - gs)
            return carry

        lax.fori_loop(0, ch * 8, step, 0)
        pltpu.sync_copy(rel_v, rel_hbm.at[pl.ds(w * ept * 4, ept * 4)])

    return pl.kernel(
        body,
        mesh=_sc_mesh(),
        compiler_params=pltpu.CompilerParams(needs_layout_passes=False),
        out_type=jax.ShapeDtypeStruct((_NW * ept * 4,), jnp.float32),
        scratch_types=[
            pltpu.VMEM((n * 3,), jnp.float32),
            pltpu.VMEM((ch, _CW), jnp.int32),
            pltpu.VMEM((ch, _CW), jnp.int32),
            pltpu.VMEM((ept * 4,), jnp.float32),
        ],
    )


def _gath_sc(n, ept, f):
    ch = ept // _CW

    def body(tab_hbm, idx_hbm, out_hbm, idx_v, buf_v, sem):
        w = _wid()
        pltpu.sync_copy(idx_hbm.at[w], idx_v)

        def step(j, carry):
            pltpu.async_copy(tab_hbm.at[idx_v.at[j]], buf_v, sem).wait()
            pltpu.sync_copy(buf_v, out_hbm.at[pl.ds(w * ept + j * _CW, _CW)])
            return carry

        lax.fori_loop(0, ch, step, 0)

    return pl.kernel(
        body,
        mesh=_sc_mesh(),
        compiler_params=pltpu.CompilerParams(needs_layout_passes=False,
                                             use_tc_tiling_on_sc=False),
        out_type=jax.ShapeDtypeStruct((_NW * ept, f), jnp.float32),
        scratch_types=[
            pltpu.VMEM((ch, _CW), jnp.int32),
            pltpu.VMEM((_CW, f), jnp.float32),
            pltpu.SemaphoreType.DMA,
        ],
    )


def _scat_sc(n, ept, f):
    ch = ept // _CW
    nrow = n + (-n % 128) + 128
    rpt = nrow // 16

    def body(msg_hbm, tgt_hbm, zero_hbm, out_hbm, idx_v, buf_v, acc_sh):
        c = lax.axis_index("c")
        s = lax.axis_index("s")
        w = s * 2 + c
        pltpu.sync_copy(tgt_hbm.at[w], idx_v)
        pltpu.sync_copy(zero_hbm.at[pl.ds(s * rpt, rpt)],
                        acc_sh.at[pl.ds(s * rpt, rpt)])
        plsc.subcore_barrier()

        def step(j, carry):
            pltpu.sync_copy(msg_hbm.at[pl.ds(w * ept + j * _CW, _CW)], buf_v)
            for k in range(_CW // 16):
                idx = idx_v[j, pl.ds(k * 16, 16)]
                pltpu.sync_copy(buf_v.at[pl.ds(k * 16, 16)],
                                acc_sh.at[idx], add=True)
            return carry

        lax.fori_loop(0, ch, step, 0)
        plsc.subcore_barrier()
        pltpu.sync_copy(acc_sh.at[pl.ds(s * rpt, rpt)],
                        out_hbm.at[c, pl.ds(s * rpt, rpt)])

    return pl.kernel(
        body,
        mesh=_sc_mesh(),
        compiler_params=pltpu.CompilerParams(needs_layout_passes=False,
                                             use_tc_tiling_on_sc=False),
        out_type=jax.ShapeDtypeStruct((2, nrow, f), jnp.float32),
        scratch_types=[
            pltpu.VMEM((ch, _CW), jnp.int32),
            pltpu.VMEM((_CW, f), jnp.float32),
            pltpu.VMEM_SHARED((nrow, f), jnp.float32),
        ],
    )


def kernel(pos, z, edge_index, batch, embed,
           freq0, rw1_0, rb1_0, rw2_0, rb2_0,
           freq1, rw1_1, rb1_1, rw2_1, rb2_1,
           freq2, rw1_2, rb1_2, rw2_2, rb2_2,
           dw1, db1, dw2, db2):
    N = pos.shape[0]
    E = edge_index.shape[1]
    NG = 16
    MAXZ = embed.shape[0]

    src = edge_index[0].astype(jnp.int32)
    tgt = edge_index[1].astype(jnp.int32)

    ep = -E % (_NW * _CW)
    E_pad = E + ep
    ept = E_pad // _NW
    src_p = jnp.concatenate([src, jnp.zeros((ep,), jnp.int32)])
    tgt_p = jnp.concatenate([tgt, jnp.full((ep,), N, jnp.int32)])
    src3 = src_p.reshape(_NW, ept // _CW, _CW)
    tgt3 = tgt_p.reshape(_NW, ept // _CW, _CW)

    eb = 1280 if E_pad % 1280 == 0 else E_pad
    nb = 1000 if N % 1000 == 0 else N

    A0 = jnp.concatenate([
        rw2_0[:, :S * (S + V)].reshape(S, S, S + V).reshape(S * S, S + V),
        rw2_0[:, S * (S + V):].reshape(S, S, V).reshape(S * S, V)], axis=1)
    B0 = jnp.concatenate([
        rb2_0[:S * (S + V)].reshape(S, S + V),
        rb2_0[S * (S + V):].reshape(S, V)], axis=1)

    A1s = rw2_1[:, :576].reshape(S, S + V, S + V).reshape(S * (S + V), S + V)
    A1sv = jnp.zeros((S, S + V, V), jnp.float32).at[:, :S, :].set(
        rw2_1[:, 576:704].reshape(S, S, V)).reshape(S * (S + V), V)
    A1 = jnp.concatenate([A1s, A1sv], axis=1)
    B1s = rb2_1[:576].reshape(S + V, S + V)
    B1sv = jnp.zeros((S + V, V), jnp.float32).at[:S, :].set(
        rb2_1[576:704].reshape(S, V))
    B1 = jnp.concatenate([B1s, B1sv], axis=1)
    A1v = rw2_1[:, 704:832].reshape(S, 2 * V, V).reshape(S * 2 * V, V)
    B1v = rb2_1[704:832].reshape(2 * V, V)

    A2 = rw2_2.reshape(S, S + V, S).reshape(S * (S + V), S)
    B2 = rb2_2.reshape(S + V, S)

    f0 = freq0.reshape(1, NB)
    f1 = freq1.reshape(1, NB)
    f2 = freq2.reshape(1, NB)
    rb1_0r = rb1_0.reshape(1, S)
    rb1_1r = rb1_1.reshape(1, S)
    rb1_2r = rb1_2.reshape(1, S)

    rel = _geom_sc(N, ept)(pos.reshape(-1), src3, tgt3).reshape(E_pad, 4)

    x0 = pl.pallas_call(
        _embed_body,
        grid=(N // nb,),
        in_specs=[pl.BlockSpec((nb, 1), lambda i: (i, 0)),
                  pl.BlockSpec((MAXZ, S), lambda i: (0, 0))],
        out_specs=pl.BlockSpec((nb, S), lambda i: (i, 0)),
        out_shape=jax.ShapeDtypeStruct((N, S), jnp.float32),
    )(z.astype(jnp.int32).reshape(N, 1), embed)

    nrow = N + (-N % 128) + 128
    zeros48 = jnp.zeros((nrow, 48), jnp.float32)
    zeros16 = jnp.zeros((nrow, 16), jnp.float32)

    def _gate(hp):
        return pl.pallas_call(
            _gate_body,
            grid=(N // nb,),
            in_specs=[pl.BlockSpec((nb, 48), lambda i: (i, 0)),
                      pl.BlockSpec((nb, 48), lambda i: (i, 0))],
            out_specs=pl.BlockSpec((nb, 48), lambda i: (i, 0)),
            out_shape=jax.ShapeDtypeStruct((N, 48), jnp.float32),
        )(hp[0, :N], hp[1, :N])

    xg = _gath_sc(N, ept, 16)(x0, src3)
    msg = _run_edge(_edge0_body, 5, rel, xg,
                    [f0, rw1_0, rb1_0r, A0, B0], 48, eb)
    x = _gate(_scat_sc(N, ept, 48)(msg, tgt3, zeros48))

    xg = _gath_sc(N, ept, 48)(x, src3)
    msg = _run_edge(_edge1_body, 7, rel, xg,
                    [f1, rw1_1, rb1_1r, A1, B1, A1v, B1v], 48, eb)
    x = _gate(_scat_sc(N, ept, 48)(msg, tgt3, zeros48))

    xg = _gath_sc(N, ept, 48)(x, src3)
    msg = _run_edge(_edge2_body, 5, rel, xg,
                    [f2, rw1_2, rb1_2r, A2, B2], 16, eb)
    hp = _scat_sc(N, ept, 16)(msg, tgt3, zeros16)

    nblocks = N // nb
    out = pl.pallas_call(
        functools.partial(_readout_body, ng=NG, nblocks=nblocks),
        grid=(nblocks,),
        in_specs=[pl.BlockSpec((nb, S), lambda i: (i, 0)),
                  pl.BlockSpec((nb, S), lambda i: (i, 0)),
                  pl.BlockSpec((nb, 1), lambda i: (i, 0)),
                  pl.BlockSpec((S, 2 * S), lambda i: (0, 0)),
                  pl.BlockSpec((1, 2 * S), lambda i: (0, 0)),
                  pl.BlockSpec((2 * S, 1), lambda i: (0, 0)),
                  pl.BlockSpec((1, 1), lambda i: (0, 0))],
        out_specs=pl.BlockSpec((1, NG), lambda i: (0, 0)),
        out_shape=jax.ShapeDtypeStruct((1, NG), jnp.float32),
        scratch_shapes=[pltpu.VMEM((2, NG), jnp.float32)],
    )(hp[0, :N], hp[1, :N], batch.astype(jnp.int32).reshape(N, 1), dw1,
      db1.reshape(1, 2 * S), dw2, db2.reshape(1, 1))
    return out[0]


def _gather_rows(table, idx):
    return table[idx]


def _segsum(vals, idx, n):
    return jax.ops.segment_sum(vals, idx, num_segments=n + 1)[:n]


# speedup vs baseline: 1.3540x; 1.0120x over previous
import functools
import numpy as np
import jax
import jax.numpy as jnp
from jax import lax
from jax.experimental import pallas as pl
from jax.experimental.pallas import tpu as pltpu
from jax.experimental.pallas import tpu_sc as plsc

S = 16
V = 8
NB = 10
CUT = 4.0

_RS = 1.0 / np.sqrt(S)
_R3 = 1.0 / np.sqrt(3.0)
_RSV = 1.0 / np.sqrt(S + V)
_RS2V = 1.0 / np.sqrt(S + 2 * V)
_R2 = 1.0 / np.sqrt(2.0)


def _sigmoid(x):
    return 1.0 / (1.0 + jnp.exp(-x))


def _silu(x):
    return x * _sigmoid(x)


def _geom(rel):
    d = jnp.sqrt(jnp.sum(rel * rel, axis=1, keepdims=True))
    dn = jnp.maximum(d, 1e-9)
    sh1 = np.float32(np.sqrt(3.0)) * rel / dn
    return d, sh1


def _radial_h(d, freq, rw1, rb1):
    x = jnp.maximum(d * np.float32(1.0 / CUT), 1e-6)
    xp = x ** 5
    env = 1.0 / x + (-28.0) * xp + 48.0 * xp * x + (-21.0) * xp * x * x
    env = jnp.where(x < 1.0, env, 0.0)
    rbf = jnp.sin(freq * x) * env
    return _silu(jnp.dot(rbf, rw1, preferred_element_type=jnp.float32) + rb1)


def _outer(h, x):
    return jnp.concatenate([h[:, k:k + 1] * x for k in range(S)], axis=1)


def _edge0_body(rel_ref, xg_ref, freq_ref, rw1_ref, rb1_ref, A_ref, Bb_ref,
                out_ref):
    d, sh1 = _geom(rel_ref[:, :3])
    h = _radial_h(d, freq_ref[...], rw1_ref[...], rb1_ref[...])
    xs = xg_ref[...]
    P = _outer(h, xs)
    o32 = (jnp.dot(P, A_ref[...], preferred_element_type=jnp.float32) +
           jnp.dot(xs, Bb_ref[...], preferred_element_type=jnp.float32))
    o32 = o32 * np.float32(_RS)
    scal = o32[:, :S + V]
    vc = o32[:, S + V:S + 2 * V]
    vecs = [vc * sh1[:, m:m + 1] for m in range(3)]
    out_ref[...] = jnp.concatenate([scal] + vecs, axis=1)


def _edge1_body(rel_ref, xg_ref, freq_ref, rw1_ref, rb1_ref, A_ref, Bb_ref,
                Av_ref, Bv_ref, out_ref):
    d, sh1 = _geom(rel_ref[:, :3])
    h = _radial_h(d, freq_ref[...], rw1_ref[...], rb1_ref[...])
    xg = xg_ref[...]
    xs = xg[:, :S]
    xv = [xg[:, S + V * m:S + V * (m + 1)] for m in range(3)]
    dot = (xv[0] * sh1[:, 0:1] + xv[1] * sh1[:, 1:2] +
           xv[2] * sh1[:, 2:3]) * np.float32(_R3)
    u = jnp.concatenate([xs, dot], axis=1)
    Q = _outer(h, u)
    o32 = (jnp.dot(Q, A_ref[...], preferred_element_type=jnp.float32) +
           jnp.dot(u, Bb_ref[...], preferred_element_type=jnp.float32))
    scal = o32[:, :S + V] * np.float32(_RSV)
    cSV = o32[:, S + V:S + 2 * V]
    crs = [
        (xv[1] * sh1[:, 2:3] - xv[2] * sh1[:, 1:2]) * np.float32(_R2),
        (xv[2] * sh1[:, 0:1] - xv[0] * sh1[:, 2:3]) * np.float32(_R2),
        (xv[0] * sh1[:, 1:2] - xv[1] * sh1[:, 0:1]) * np.float32(_R2),
    ]
    Av = Av_ref[...]
    Bv = Bv_ref[...]
    vecs = []
    for m in range(3):
        G = jnp.concatenate([xv[m], crs[m]], axis=1)
        R = _outer(h, G)
        v0c = (jnp.dot(R, Av, preferred_element_type=jnp.float32) +
               jnp.dot(G, Bv, preferred_element_type=jnp.float32))
        vecs.append((cSV * sh1[:, m:m + 1] + v0c) * np.float32(_RS2V))
    out_ref[...] = jnp.concatenate([scal] + vecs, axis=1)


def _edge2_body(rel_ref, xg_ref, freq_ref, rw1_ref, rb1_ref, A_ref, Bb_ref,
                out_ref):
    d, sh1 = _geom(rel_ref[:, :3])
    h = _radial_h(d, freq_ref[...], rw1_ref[...], rb1_ref[...])
    xg = xg_ref[...]
    xs = xg[:, :S]
    xv = [xg[:, S + V * m:S + V * (m + 1)] for m in range(3)]
    dot = (xv[0] * sh1[:, 0:1] + xv[1] * sh1[:, 1:2] +
           xv[2] * sh1[:, 2:3]) * np.float32(_R3)
    u = jnp.concatenate([xs, dot], axis=1)
    Q = _outer(h, u)
    out_ref[...] = (jnp.dot(Q, A_ref[...], preferred_element_type=jnp.float32)
                    + jnp.dot(u, Bb_ref[...],
                              preferred_element_type=jnp.float32)
                    ) * np.float32(_RSV)


def _run_edge(body, n_extra, rel, xg, consts, out_dim, eb):
    E = rel.shape[0]
    grid = E // eb
    full = lambda a: pl.BlockSpec(a.shape, lambda i: (0,) * a.ndim)
    in_specs = [
        pl.BlockSpec((eb, rel.shape[1]), lambda i: (i, 0)),
        pl.BlockSpec((eb, xg.shape[1]), lambda i: (i, 0)),
    ] + [full(c) for c in consts]
    return pl.pallas_call(
        body,
        grid=(grid,),
        in_specs=in_specs,
        out_specs=pl.BlockSpec((eb, out_dim), lambda i: (i, 0)),
        out_shape=jax.ShapeDtypeStruct((E, out_dim), jnp.float32),
    )(rel, xg, *consts)


def _embed_body(z_ref, embed_ref, out_ref):
    z = z_ref[...]
    emb = embed_ref[...]
    acc = jnp.zeros((z.shape[0], S), jnp.float32)
    for c in range(emb.shape[0]):
        acc = acc + jnp.where(z == c, 1.0, 0.0) * emb[c][None, :]
    out_ref[...] = acc


def _gate_body(ha_ref, hb_ref, out_ref):
    h = ha_ref[...] + hb_ref[...]
    scal = _silu(h[:, :S])
    g = _sigmoid(h[:, S:S + V])
    vecs = [h[:, S + V + V * m:S + V + V * (m + 1)] * g for m in range(3)]
    pad = jnp.zeros((h.shape[0], V), h.dtype)
    out_ref[...] = jnp.concatenate([scal] + vecs + [pad], axis=1)


def _readout_body(ha_ref, hb_ref, b_ref, dw1_ref, db1_ref, dw2_ref, db2_ref,
                  out_ref, acc_ref, *, ng, nblocks):
    i = pl.program_id(0)

    @pl.when(i == 0)
    def _init():
        acc_ref[...] = jnp.zeros_like(acc_ref)

    h = ha_ref[...] + hb_ref[...]
    t = jax.nn.relu(jnp.dot(h, dw1_ref[...],
                            preferred_element_type=jnp.float32) + db1_ref[...])
    y = jnp.dot(t, dw2_ref[...], preferred_element_type=jnp.float32) \
        + db2_ref[...]
    b = b_ref[...]
    gid = jax.lax.broadcasted_iota(jnp.int32, (1, ng), 1)
    onehot = jnp.where(b == gid, 1.0, 0.0)
    sums = jnp.sum(onehot * y, axis=0, keepdims=True)
    cnts = jnp.sum(onehot, axis=0, keepdims=True)
    acc_ref[0:1, :] += sums
    acc_ref[1:2, :] += cnts

    @pl.when(i == nblocks - 1)
    def _fin():
        out_ref[...] = acc_ref[0:1, :] / jnp.maximum(acc_ref[1:2, :], 1.0)


_NW = 32
_CW = 128


def _sc_mesh():
    return plsc.VectorSubcoreMesh(core_axis_name="c", subcore_axis_name="s")


def _wid():
    return lax.axis_index("s") * 2 + lax.axis_index("c")


def _geom_sc(n, ept):
    ch = ept // _CW

    def body(pos_hbm, src_hbm, tgt_hbm, rel_hbm, pos_v, si_v, ti_v, rel_v):
        w = _wid()
        pltpu.sync_copy(pos_hbm, pos_v)
        pltpu.sync_copy(src_hbm.at[w], si_v)
        pltpu.sync_copy(tgt_hbm.at[w], ti_v)
        iota = lax.broadcasted_iota(jnp.int32, (16,), 0)

        def step(i, carry):
            j = i // 8
            k = i % 8
            sl = pl.ds(k * 16, 16)
            si = si_v[j, sl] * 3
            ti = ti_v[j, sl] * 3
            row = (i * 16 + iota) * 4
            for c in range(3):
                gs = plsc.load_gather(pos_v, [si + c])
                gt = plsc.load_gather(pos_v, [ti + c])
                plsc.store_scatter(rel_v, [row + c], gt - gs)
            return carry

        lax.fori_loop(0, ch * 8, step, 0)
        pltpu.sync_copy(rel_v, rel_hbm.at[pl.ds(w * ept * 4, ept * 4)])

    return pl.kernel(
        body,
        mesh=_sc_mesh(),
        compiler_params=pltpu.CompilerParams(needs_layout_passes=False),
        out_type=jax.ShapeDtypeStruct((_NW * ept * 4,), jnp.float32),
        scratch_types=[
            pltpu.VMEM((n * 3,), jnp.float32),
            pltpu.VMEM((ch, _CW), jnp.int32),
            pltpu.VMEM((ch, _CW), jnp.int32),
            pltpu.VMEM((ept * 4,), jnp.float32),
        ],
    )


def _gath_sc(n, ept, f):
    ch = ept // _CW

    def body(tab_hbm, idx_hbm, out_hbm, idx_v, buf_v, sem):
        w = _wid()
        pltpu.sync_copy(idx_hbm.at[w], idx_v)

        def step(j, carry):
            pltpu.async_copy(tab_hbm.at[idx_v.at[j]], buf_v, sem).wait()
            pltpu.sync_copy(buf_v, out_hbm.at[pl.ds(w * ept + j * _CW, _CW)])
            return carry

        lax.fori_loop(0, ch, step, 0)

    return pl.kernel(
        body,
        mesh=_sc_mesh(),
        compiler_params=pltpu.CompilerParams(needs_layout_passes=False,
                                             use_tc_tiling_on_sc=False),
        out_type=jax.ShapeDtypeStruct((_NW * ept, f), jnp.float32),
        scratch_types=[
            pltpu.VMEM((ch, _CW), jnp.int32),
            pltpu.VMEM((_CW, f), jnp.float32),
            pltpu.SemaphoreType.DMA,
        ],
    )


def _scat_sc(n, ept, f):
    ch = ept // _CW
    nrow = n + (-n % 128) + 128
    rpt = nrow // 16

    def body(msg_hbm, tgt_hbm, zero_hbm, out_hbm, idx_v, buf_v, acc_sh):
        c = lax.axis_index("c")
        s = lax.axis_index("s")
        w = s * 2 + c
        pltpu.sync_copy(tgt_hbm.at[w], idx_v)
        pltpu.sync_copy(zero_hbm.at[pl.ds(s * rpt, rpt)],
                        acc_sh.at[pl.ds(s * rpt, rpt)])
        plsc.subcore_barrier()

        def step(j, carry):
            pltpu.sync_copy(msg_hbm.at[pl.ds(w * ept + j * _CW, _CW)], buf_v)
            for k in range(_CW // 16):
                idx = idx_v[j, pl.ds(k * 16, 16)]
                pltpu.sync_copy(buf_v.at[pl.ds(k * 16, 16)],
                                acc_sh.at[idx], add=True)
            return carry

        lax.fori_loop(0, ch, step, 0)
        plsc.subcore_barrier()
        pltpu.sync_copy(acc_sh.at[pl.ds(s * rpt, rpt)],
                        out_hbm.at[c, pl.ds(s * rpt, rpt)])

    return pl.kernel(
        body,
        mesh=_sc_mesh(),
        compiler_params=pltpu.CompilerParams(needs_layout_passes=False,
                                             use_tc_tiling_on_sc=False),
        out_type=jax.ShapeDtypeStruct((2, nrow, f), jnp.float32),
        scratch_types=[
            pltpu.VMEM((ch, _CW), jnp.int32),
            pltpu.VMEM((_CW, f), jnp.float32),
            pltpu.VMEM_SHARED((nrow, f), jnp.float32),
        ],
    )


def kernel(pos, z, edge_index, batch, embed,
           freq0, rw1_0, rb1_0, rw2_0, rb2_0,
           freq1, rw1_1, rb1_1, rw2_1, rb2_1,
           freq2, rw1_2, rb1_2, rw2_2, rb2_2,
           dw1, db1, dw2, db2):
    N = pos.shape[0]
    E = edge_index.shape[1]
    NG = 16
    MAXZ = embed.shape[0]

    src = edge_index[0].astype(jnp.int32)
    tgt = edge_index[1].astype(jnp.int32)

    ep = -E % (_NW * _CW)
    E_pad = E + ep
    ept = E_pad // _NW
    src_p = jnp.concatenate([src, jnp.zeros((ep,), jnp.int32)])
    tgt_p = jnp.concatenate([tgt, jnp.full((ep,), N, jnp.int32)])
    src3 = src_p.reshape(_NW, ept // _CW, _CW)
    tgt3 = tgt_p.reshape(_NW, ept // _CW, _CW)

    eb = 2560 if E_pad % 2560 == 0 else E_pad
    nb = 1000 if N % 1000 == 0 else N

    A0 = jnp.concatenate([
        rw2_0[:, :S * (S + V)].reshape(S, S, S + V).reshape(S * S, S + V),
        rw2_0[:, S * (S + V):].reshape(S, S, V).reshape(S * S, V)], axis=1)
    B0 = jnp.concatenate([
        rb2_0[:S * (S + V)].reshape(S, S + V),
        rb2_0[S * (S + V):].reshape(S, V)], axis=1)

    A1s = rw2_1[:, :576].reshape(S, S + V, S + V).reshape(S * (S + V), S + V)
    A1sv = jnp.zeros((S, S + V, V), jnp.float32).at[:, :S, :].set(
        rw2_1[:, 576:704].reshape(S, S, V)).reshape(S * (S + V), V)
    A1 = jnp.concatenate([A1s, A1sv], axis=1)
    B1s = rb2_1[:576].reshape(S + V, S + V)
    B1sv = jnp.zeros((S + V, V), jnp.float32).at[:S, :].set(
        rb2_1[576:704].reshape(S, V))
    B1 = jnp.concatenate([B1s, B1sv], axis=1)
    A1v = rw2_1[:, 704:832].reshape(S, 2 * V, V).reshape(S * 2 * V, V)
    B1v = rb2_1[704:832].reshape(2 * V, V)

    A2 = rw2_2.reshape(S, S + V, S).reshape(S * (S + V), S)
    B2 = rb2_2.reshape(S + V, S)

    f0 = freq0.reshape(1, NB)
    f1 = freq1.reshape(1, NB)
    f2 = freq2.reshape(1, NB)
    rb1_0r = rb1_0.reshape(1, S)
    rb1_1r = rb1_1.reshape(1, S)
    rb1_2r = rb1_2.reshape(1, S)

    rel = _geom_sc(N, ept)(pos.reshape(-1), src3, tgt3).reshape(E_pad, 4)

    x0 = pl.pallas_call(
        _embed_body,
        grid=(N // nb,),
        in_specs=[pl.BlockSpec((nb, 1), lambda i: (i, 0)),
                  pl.BlockSpec((MAXZ, S), lambda i: (0, 0))],
        out_specs=pl.BlockSpec((nb, S), lambda i: (i, 0)),
        out_shape=jax.ShapeDtypeStruct((N, S), jnp.float32),
    )(z.astype(jnp.int32).reshape(N, 1), embed)

    nrow = N + (-N % 128) + 128
    zeros48 = jnp.zeros((nrow, 48), jnp.float32)
    zeros16 = jnp.zeros((nrow, 16), jnp.float32)

    def _gate(hp):
        return pl.pallas_call(
            _gate_body,
            grid=(N // nb,),
            in_specs=[pl.BlockSpec((nb, 48), lambda i: (i, 0)),
                      pl.BlockSpec((nb, 48), lambda i: (i, 0))],
            out_specs=pl.BlockSpec((nb, 48), lambda i: (i, 0)),
            out_shape=jax.ShapeDtypeStruct((N, 48), jnp.float32),
        )(hp[0, :N], hp[1, :N])

    xg = _gath_sc(N, ept, 16)(x0, src3)
    msg = _run_edge(_edge0_body, 5, rel, xg,
                    [f0, rw1_0, rb1_0r, A0, B0], 48, eb)
    x = _gate(_scat_sc(N, ept, 48)(msg, tgt3, zeros48))

    xg = _gath_sc(N, ept, 48)(x, src3)
    msg = _run_edge(_edge1_body, 7, rel, xg,
                    [f1, rw1_1, rb1_1r, A1, B1, A1v, B1v], 48, eb)
    x = _gate(_scat_sc(N, ept, 48)(msg, tgt3, zeros48))

    xg = _gath_sc(N, ept, 48)(x, src3)
    msg = _run_edge(_edge2_body, 5, rel, xg,
                    [f2, rw1_2, rb1_2r, A2, B2], 16, eb)
    hp = _scat_sc(N, ept, 16)(msg, tgt3, zeros16)

    nblocks = N // nb
    out = pl.pallas_call(
        functools.partial(_readout_body, ng=NG, nblocks=nblocks),
        grid=(nblocks,),
        in_specs=[pl.BlockSpec((nb, S), lambda i: (i, 0)),
                  pl.BlockSpec((nb, S), lambda i: (i, 0)),
                  pl.BlockSpec((nb, 1), lambda i: (i, 0)),
                  pl.BlockSpec((S, 2 * S), lambda i: (0, 0)),
                  pl.BlockSpec((1, 2 * S), lambda i: (0, 0)),
                  pl.BlockSpec((2 * S, 1), lambda i: (0, 0)),
                  pl.BlockSpec((1, 1), lambda i: (0, 0))],
        out_specs=pl.BlockSpec((1, NG), lambda i: (0, 0)),
        out_shape=jax.ShapeDtypeStruct((1, NG), jnp.float32),
        scratch_shapes=[pltpu.VMEM((2, NG), jnp.float32)],
    )(hp[0, :N], hp[1, :N], batch.astype(jnp.int32).reshape(N, 1), dw1,
      db1.reshape(1, 2 * S), dw2, db2.reshape(1, 1))
    return out[0]


def _gather_rows(table, idx):
    return table[idx]


def _segsum(vals, idx, n):
    return jax.ops.segment_sum(vals, idx, num_segments=n + 1)[:n]


# speedup vs baseline: 2.2893x; 1.6907x over previous
import functools
import numpy as np
import jax
import jax.numpy as jnp
from jax import lax
from jax.experimental import pallas as pl
from jax.experimental.pallas import tpu as pltpu
from jax.experimental.pallas import tpu_sc as plsc

S = 16
V = 8
NB = 10
CUT = 4.0

_RS = 1.0 / np.sqrt(S)
_R3 = 1.0 / np.sqrt(3.0)
_RSV = 1.0 / np.sqrt(S + V)
_RS2V = 1.0 / np.sqrt(S + 2 * V)
_R2 = 1.0 / np.sqrt(2.0)


def _sigmoid(x):
    return 1.0 / (1.0 + jnp.exp(-x))


def _silu(x):
    return x * _sigmoid(x)


def _geom(rel):
    d = jnp.sqrt(jnp.sum(rel * rel, axis=1, keepdims=True))
    dn = jnp.maximum(d, 1e-9)
    sh1 = np.float32(np.sqrt(3.0)) * rel / dn
    return d, sh1


def _radial_h(d, freq, rw1, rb1):
    x = jnp.maximum(d * np.float32(1.0 / CUT), 1e-6)
    xp = x ** 5
    env = 1.0 / x + (-28.0) * xp + 48.0 * xp * x + (-21.0) * xp * x * x
    env = jnp.where(x < 1.0, env, 0.0)
    rbf = jnp.sin(freq * x) * env
    return _silu(jnp.dot(rbf, rw1, preferred_element_type=jnp.float32) + rb1)


def _fmm(x, h, a2d_ref, rh_ref, g_ref):
    t = jnp.dot(x, a2d_ref[...], preferred_element_type=jnp.float32)
    hr = jnp.dot(h, rh_ref[...], preferred_element_type=jnp.float32)
    return jnp.dot(t * hr, g_ref[...], preferred_element_type=jnp.float32)


def _edge0_body(rel_ref, xg_ref, freq_ref, rw1_ref, rb1_ref, a2d_ref, rh_ref,
                g_ref, Bb_ref, out_ref):
    d, sh1 = _geom(rel_ref[:, :3])
    h = _radial_h(d, freq_ref[...], rw1_ref[...], rb1_ref[...])
    xs = xg_ref[...]
    o32 = (_fmm(xs, h, a2d_ref, rh_ref, g_ref) +
           jnp.dot(xs, Bb_ref[...], preferred_element_type=jnp.float32))
    o32 = o32 * np.float32(_RS)
    scal = o32[:, :S + V]
    vc = o32[:, S + V:S + 2 * V]
    vecs = [vc * sh1[:, m:m + 1] for m in range(3)]
    out_ref[...] = jnp.concatenate([scal] + vecs, axis=1)


def _edge1_body(rel_ref, xg_ref, freq_ref, rw1_ref, rb1_ref, a2d_ref,
                a2dv_ref, rh32_ref, rh8_ref, g32_ref, g8_ref, Bb_ref, Bv_ref,
                out_ref):
    d, sh1 = _geom(rel_ref[:, :3])
    h = _radial_h(d, freq_ref[...], rw1_ref[...], rb1_ref[...])
    xg = xg_ref[...]
    xs = xg[:, :S]
    xv = [xg[:, S + V * m:S + V * (m + 1)] for m in range(3)]
    dot = (xv[0] * sh1[:, 0:1] + xv[1] * sh1[:, 1:2] +
           xv[2] * sh1[:, 2:3]) * np.float32(_R3)
    u = jnp.concatenate([xs, dot], axis=1)
    o32 = (_fmm(u, h, a2d_ref, rh32_ref, g32_ref) +
           jnp.dot(u, Bb_ref[...], preferred_element_type=jnp.float32))
    scal = o32[:, :S + V] * np.float32(_RSV)
    cSV = o32[:, S + V:S + 2 * V]
    crs = [
        (xv[1] * sh1[:, 2:3] - xv[2] * sh1[:, 1:2]) * np.float32(_R2),
        (xv[2] * sh1[:, 0:1] - xv[0] * sh1[:, 2:3]) * np.float32(_R2),
        (xv[0] * sh1[:, 1:2] - xv[1] * sh1[:, 0:1]) * np.float32(_R2),
    ]
    hr8 = jnp.dot(h, rh8_ref[...], preferred_element_type=jnp.float32)
    Av = a2dv_ref[...]
    g8 = g8_ref[...]
    Bv = Bv_ref[...]
    vecs = []
    for m in range(3):
        G = jnp.concatenate([xv[m], crs[m]], axis=1)
        tv = jnp.dot(G, Av, preferred_element_type=jnp.float32)
        v0c = (jnp.dot(tv * hr8, g8, preferred_element_type=jnp.float32) +
               jnp.dot(G, Bv, preferred_element_type=jnp.float32))
        vecs.append((cSV * sh1[:, m:m + 1] + v0c) * np.float32(_RS2V))
    out_ref[...] = jnp.concatenate([scal] + vecs, axis=1)


def _edge2_body(rel_ref, xg_ref, freq_ref, rw1_ref, rb1_ref, a2d_ref, rh_ref,
                g_ref, Bb_ref, out_ref):
    d, sh1 = _geom(rel_ref[:, :3])
    h = _radial_h(d, freq_ref[...], rw1_ref[...], rb1_ref[...])
    xg = xg_ref[...]
    xs = xg[:, :S]
    xv = [xg[:, S + V * m:S + V * (m + 1)] for m in range(3)]
    dot = (xv[0] * sh1[:, 0:1] + xv[1] * sh1[:, 1:2] +
           xv[2] * sh1[:, 2:3]) * np.float32(_R3)
    u = jnp.concatenate([xs, dot], axis=1)
    out_ref[...] = (_fmm(u, h, a2d_ref, rh_ref, g_ref) +
                    jnp.dot(u, Bb_ref[...],
                            preferred_element_type=jnp.float32)
                    ) * np.float32(_RSV)


def _run_edge(body, n_extra, rel, xg, consts, out_dim, eb):
    E = rel.shape[0]
    grid = E // eb
    full = lambda a: pl.BlockSpec(a.shape, lambda i: (0,) * a.ndim)
    in_specs = [
        pl.BlockSpec((eb, rel.shape[1]), lambda i: (i, 0)),
        pl.BlockSpec((eb, xg.shape[1]), lambda i: (i, 0)),
    ] + [full(c) for c in consts]
    return pl.pallas_call(
        body,
        grid=(grid,),
        in_specs=in_specs,
        out_specs=pl.BlockSpec((eb, out_dim), lambda i: (i, 0)),
        out_shape=jax.ShapeDtypeStruct((E, out_dim), jnp.float32),
    )(rel, xg, *consts)


def _embed_body(z_ref, embed_ref, out_ref):
    z = z_ref[...]
    emb = embed_ref[...]
    acc = jnp.zeros((z.shape[0], S), jnp.float32)
    for c in range(emb.shape[0]):
        acc = acc + jnp.where(z == c, 1.0, 0.0) * emb[c][None, :]
    out_ref[...] = acc


def _gate_body(ha_ref, hb_ref, out_ref):
    h = ha_ref[...] + hb_ref[...]
    scal = _silu(h[:, :S])
    g = _sigmoid(h[:, S:S + V])
    vecs = [h[:, S + V + V * m:S + V + V * (m + 1)] * g for m in range(3)]
    pad = jnp.zeros((h.shape[0], V), h.dtype)
    out_ref[...] = jnp.concatenate([scal] + vecs + [pad], axis=1)


def _readout_body(ha_ref, hb_ref, b_ref, dw1_ref, db1_ref, dw2_ref, db2_ref,
                  out_ref, acc_ref, *, ng, nblocks):
    i = pl.program_id(0)

    @pl.when(i == 0)
    def _init():
        acc_ref[...] = jnp.zeros_like(acc_ref)

    h = ha_ref[...] + hb_ref[...]
    t = jax.nn.relu(jnp.dot(h, dw1_ref[...],
                            preferred_element_type=jnp.float32) + db1_ref[...])
    y = jnp.dot(t, dw2_ref[...], preferred_element_type=jnp.float32) \
        + db2_ref[...]
    b = b_ref[...]
    gid = jax.lax.broadcasted_iota(jnp.int32, (1, ng), 1)
    onehot = jnp.where(b == gid, 1.0, 0.0)
    sums = jnp.sum(onehot * y, axis=0, keepdims=True)
    cnts = jnp.sum(onehot, axis=0, keepdims=True)
    acc_ref[0:1, :] += sums
    acc_ref[1:2, :] += cnts

    @pl.when(i == nblocks - 1)
    def _fin():
        out_ref[...] = acc_ref[0:1, :] / jnp.maximum(acc_ref[1:2, :], 1.0)


_NW = 32
_CW = 128


def _sc_mesh():
    return plsc.VectorSubcoreMesh(core_axis_name="c", subcore_axis_name="s")


def _wid():
    return lax.axis_index("s") * 2 + lax.axis_index("c")


def _geom_sc(n, ept):
    ch = ept // _CW

    def body(pos_hbm, src_hbm, tgt_hbm, rel_hbm, pos_v, si_v, ti_v, rel_v):
        w = _wid()
        pltpu.sync_copy(pos_hbm, pos_v)
        pltpu.sync_copy(src_hbm.at[w], si_v)
        pltpu.sync_copy(tgt_hbm.at[w], ti_v)
        iota = lax.broadcasted_iota(jnp.int32, (16,), 0)

        def step(i, carry):
            j = i // 8
            k = i % 8
            sl = pl.ds(k * 16, 16)
            si = si_v[j, sl] * 3
            ti = ti_v[j, sl] * 3
            row = (i * 16 + iota) * 4
            for c in range(3):
                gs = plsc.load_gather(pos_v, [si + c])
                gt = plsc.load_gather(pos_v, [ti + c])
                plsc.store_scatter(rel_v, [row + c], gt - gs)
            return carry

        lax.fori_loop(0, ch * 8, step, 0)
        pltpu.sync_copy(rel_v, rel_hbm.at[pl.ds(w * ept * 4, ept * 4)])

    return pl.kernel(
        body,
        mesh=_sc_mesh(),
        compiler_params=pltpu.CompilerParams(needs_layout_passes=False),
        out_type=jax.ShapeDtypeStruct((_NW * ept * 4,), jnp.float32),
        scratch_types=[
            pltpu.VMEM((n * 3,), jnp.float32),
            pltpu.VMEM((ch, _CW), jnp.int32),
            pltpu.VMEM((ch, _CW), jnp.int32),
            pltpu.VMEM((ept * 4,), jnp.float32),
        ],
    )


def _gath_sc(n, ept, f):
    ch = ept // _CW

    def body(tab_hbm, idx_hbm, out_hbm, idx_v, buf_v, sem):
        w = _wid()
        pltpu.sync_copy(idx_hbm.at[w], idx_v)

        def step(j, carry):
            pltpu.async_copy(tab_hbm.at[idx_v.at[j]], buf_v, sem).wait()
            pltpu.sync_copy(buf_v, out_hbm.at[pl.ds(w * ept + j * _CW, _CW)])
            return carry

        lax.fori_loop(0, ch, step, 0)

    return pl.kernel(
        body,
        mesh=_sc_mesh(),
        compiler_params=pltpu.CompilerParams(needs_layout_passes=False,
                                             use_tc_tiling_on_sc=False),
        out_type=jax.ShapeDtypeStruct((_NW * ept, f), jnp.float32),
        scratch_types=[
            pltpu.VMEM((ch, _CW), jnp.int32),
            pltpu.VMEM((_CW, f), jnp.float32),
            pltpu.SemaphoreType.DMA,
        ],
    )


def _scat_sc(n, ept, f):
    ch = ept // _CW
    nrow = n + (-n % 128) + 128
    rpt = nrow // 16

    def body(msg_hbm, tgt_hbm, zero_hbm, out_hbm, idx_v, buf_v, acc_sh):
        c = lax.axis_index("c")
        s = lax.axis_index("s")
        w = s * 2 + c
        pltpu.sync_copy(tgt_hbm.at[w], idx_v)
        pltpu.sync_copy(zero_hbm.at[pl.ds(s * rpt, rpt)],
                        acc_sh.at[pl.ds(s * rpt, rpt)])
        plsc.subcore_barrier()

        def step(j, carry):
            pltpu.sync_copy(msg_hbm.at[pl.ds(w * ept + j * _CW, _CW)], buf_v)
            for k in range(_CW // 16):
                idx = idx_v[j, pl.ds(k * 16, 16)]
                pltpu.sync_copy(buf_v.at[pl.ds(k * 16, 16)],
                                acc_sh.at[idx], add=True)
            return carry

        lax.fori_loop(0, ch, step, 0)
        plsc.subcore_barrier()
        pltpu.sync_copy(acc_sh.at[pl.ds(s * rpt, rpt)],
                        out_hbm.at[c, pl.ds(s * rpt, rpt)])

    return pl.kernel(
        body,
        mesh=_sc_mesh(),
        compiler_params=pltpu.CompilerParams(needs_layout_passes=False,
                                             use_tc_tiling_on_sc=False),
        out_type=jax.ShapeDtypeStruct((2, nrow, f), jnp.float32),
        scratch_types=[
            pltpu.VMEM((ch, _CW), jnp.int32),
            pltpu.VMEM((_CW, f), jnp.float32),
            pltpu.VMEM_SHARED((nrow, f), jnp.float32),
        ],
    )


def kernel(pos, z, edge_index, batch, embed,
           freq0, rw1_0, rb1_0, rw2_0, rb2_0,
           freq1, rw1_1, rb1_1, rw2_1, rb2_1,
           freq2, rw1_2, rb1_2, rw2_2, rb2_2,
           dw1, db1, dw2, db2):
    N = pos.shape[0]
    E = edge_index.shape[1]
    NG = 16
    MAXZ = embed.shape[0]

    src = edge_index[0].astype(jnp.int32)
    tgt = edge_index[1].astype(jnp.int32)

    ep = -E % (_NW * _CW)
    E_pad = E + ep
    ept = E_pad // _NW
    src_p = jnp.concatenate([src, jnp.zeros((ep,), jnp.int32)])
    tgt_p = jnp.concatenate([tgt, jnp.full((ep,), N, jnp.int32)])
    src3 = src_p.reshape(_NW, ept // _CW, _CW)
    tgt3 = tgt_p.reshape(_NW, ept // _CW, _CW)

    eb = 2560 if E_pad % 2560 == 0 else E_pad
    nb = 1000 if N % 1000 == 0 else N

    A0 = jnp.concatenate([
        rw2_0[:, :S * (S + V)].reshape(S, S, S + V).reshape(S * S, S + V),
        rw2_0[:, S * (S + V):].reshape(S, S, V).reshape(S * S, V)], axis=1)
    B0 = jnp.concatenate([
        rb2_0[:S * (S + V)].reshape(S, S + V),
        rb2_0[S * (S + V):].reshape(S, V)], axis=1)

    A1s = rw2_1[:, :576].reshape(S, S + V, S + V).reshape(S * (S + V), S + V)
    A1sv = jnp.zeros((S, S + V, V), jnp.float32).at[:, :S, :].set(
        rw2_1[:, 576:704].reshape(S, S, V)).reshape(S * (S + V), V)
    A1 = jnp.concatenate([A1s, A1sv], axis=1)
    B1s = rb2_1[:576].reshape(S + V, S + V)
    B1sv = jnp.zeros((S + V, V), jnp.float32).at[:S, :].set(
        rb2_1[576:704].reshape(S, V))
    B1 = jnp.concatenate([B1s, B1sv], axis=1)
    A1v = rw2_1[:, 704:832].reshape(S, 2 * V, V).reshape(S * 2 * V, V)
    B1v = rb2_1[704:832].reshape(2 * V, V)

    A2 = rw2_2.reshape(S, S + V, S).reshape(S * (S + V), S)
    B2 = rb2_2.reshape(S + V, S)

    def _a2d(a, f, o):
        return a.reshape(S, f, o).transpose(1, 0, 2).reshape(f, S * o)

    def _rh(o):
        return jnp.asarray(np.kron(np.eye(S), np.ones((1, o))), jnp.float32)

    def _gm(o):
        return jnp.asarray(np.tile(np.eye(o), (S, 1)), jnp.float32)

    A2d0 = _a2d(A0, S, 2 * S)
    A2d1 = _a2d(A1, S + V, 2 * S)
    A2dv = _a2d(A1v, 2 * V, V)
    A2d2 = _a2d(A2, S + V, S)
    RH32, RH16, RH8 = _rh(2 * S), _rh(S), _rh(V)
    G32, G16, G8 = _gm(2 * S), _gm(S), _gm(V)

    f0 = freq0.reshape(1, NB)
    f1 = freq1.reshape(1, NB)
    f2 = freq2.reshape(1, NB)
    rb1_0r = rb1_0.reshape(1, S)
    rb1_1r = rb1_1.reshape(1, S)
    rb1_2r = rb1_2.reshape(1, S)

    rel = _geom_sc(N, ept)(pos.reshape(-1), src3, tgt3).reshape(E_pad, 4)

    x0 = pl.pallas_call(
        _embed_body,
        grid=(N // nb,),
        in_specs=[pl.BlockSpec((nb, 1), lambda i: (i, 0)),
                  pl.BlockSpec((MAXZ, S), lambda i: (0, 0))],
        out_specs=pl.BlockSpec((nb, S), lambda i: (i, 0)),
        out_shape=jax.ShapeDtypeStruct((N, S), jnp.float32),
    )(z.astype(jnp.int32).reshape(N, 1), embed)

    nrow = N + (-N % 128) + 128
    zeros48 = jnp.zeros((nrow, 48), jnp.float32)
    zeros16 = jnp.zeros((nrow, 16), jnp.float32)

    def _gate(hp):
        return pl.pallas_call(
            _gate_body,
            grid=(N // nb,),
            in_specs=[pl.BlockSpec((nb, 48), lambda i: (i, 0)),
                      pl.BlockSpec((nb, 48), lambda i: (i, 0))],
            out_specs=pl.BlockSpec((nb, 48), lambda i: (i, 0)),
            out_shape=jax.ShapeDtypeStruct((N, 48), jnp.float32),
        )(hp[0, :N], hp[1, :N])

    xg = _gath_sc(N, ept, 16)(x0, src3)
    msg = _run_edge(_edge0_body, 5, rel, xg,
                    [f0, rw1_0, rb1_0r, A2d0, RH32, G32, B0], 48, eb)
    x = _gate(_scat_sc(N, ept, 48)(msg, tgt3, zeros48))

    xg = _gath_sc(N, ept, 48)(x, src3)
    msg = _run_edge(_edge1_body, 7, rel, xg,
                    [f1, rw1_1, rb1_1r, A2d1, A2dv, RH32, RH8, G32, G8,
                     B1, B1v], 48, eb)
    x = _gate(_scat_sc(N, ept, 48)(msg, tgt3, zeros48))

    xg = _gath_sc(N, ept, 48)(x, src3)
    msg = _run_edge(_edge2_body, 5, rel, xg,
                    [f2, rw1_2, rb1_2r, A2d2, RH16, G16, B2], 16, eb)
    hp = _scat_sc(N, ept, 16)(msg, tgt3, zeros16)

    nblocks = N // nb
    out = pl.pallas_call(
        functools.partial(_readout_body, ng=NG, nblocks=nblocks),
        grid=(nblocks,),
        in_specs=[pl.BlockSpec((nb, S), lambda i: (i, 0)),
                  pl.BlockSpec((nb, S), lambda i: (i, 0)),
                  pl.BlockSpec((nb, 1), lambda i: (i, 0)),
                  pl.BlockSpec((S, 2 * S), lambda i: (0, 0)),
                  pl.BlockSpec((1, 2 * S), lambda i: (0, 0)),
                  pl.BlockSpec((2 * S, 1), lambda i: (0, 0)),
                  pl.BlockSpec((1, 1), lambda i: (0, 0))],
        out_specs=pl.BlockSpec((1, NG), lambda i: (0, 0)),
        out_shape=jax.ShapeDtypeStruct((1, NG), jnp.float32),
        scratch_shapes=[pltpu.VMEM((2, NG), jnp.float32)],
    )(hp[0, :N], hp[1, :N], batch.astype(jnp.int32).reshape(N, 1), dw1,
      db1.reshape(1, 2 * S), dw2, db2.reshape(1, 1))
    return out[0]


def _gather_rows(table, idx):
    return table[idx]


def _segsum(vals, idx, n):
    return jax.ops.segment_sum(vals, idx, num_segments=n + 1)[:n]


# speedup vs baseline: 2.3246x; 1.0154x over previous
import functools
import numpy as np
import jax
import jax.numpy as jnp
from jax import lax
from jax.experimental import pallas as pl
from jax.experimental.pallas import tpu as pltpu
from jax.experimental.pallas import tpu_sc as plsc

S = 16
V = 8
NB = 10
CUT = 4.0

_RS = 1.0 / np.sqrt(S)
_R3 = 1.0 / np.sqrt(3.0)
_RSV = 1.0 / np.sqrt(S + V)
_RS2V = 1.0 / np.sqrt(S + 2 * V)
_R2 = 1.0 / np.sqrt(2.0)


def _sigmoid(x):
    return 1.0 / (1.0 + jnp.exp(-x))


def _silu(x):
    return x * _sigmoid(x)


def _geom(rel):
    d = jnp.sqrt(jnp.sum(rel * rel, axis=1, keepdims=True))
    dn = jnp.maximum(d, 1e-9)
    sh1 = np.float32(np.sqrt(3.0)) * rel / dn
    return d, sh1


def _radial_h(d, freq, rw1, rb1):
    x = jnp.maximum(d * np.float32(1.0 / CUT), 1e-6)
    xp = x ** 5
    env = 1.0 / x + (-28.0) * xp + 48.0 * xp * x + (-21.0) * xp * x * x
    env = jnp.where(x < 1.0, env, 0.0)
    rbf = jnp.sin(freq * x) * env
    return _silu(jnp.dot(rbf, rw1, preferred_element_type=jnp.float32) + rb1)


def _fmm(x, h, a2d_ref, rh_ref, g_ref):
    t = jnp.dot(x, a2d_ref[...], preferred_element_type=jnp.float32)
    hr = jnp.dot(h, rh_ref[...], preferred_element_type=jnp.float32)
    return jnp.dot(t * hr, g_ref[...], preferred_element_type=jnp.float32)


def _edge0_body(rel_ref, xg_ref, freq_ref, rw1_ref, rb1_ref, a2d_ref, rh_ref,
                g_ref, Bb_ref, out_ref):
    d, sh1 = _geom(rel_ref[:, :3])
    h = _radial_h(d, freq_ref[...], rw1_ref[...], rb1_ref[...])
    xs = xg_ref[...]
    o32 = (_fmm(xs, h, a2d_ref, rh_ref, g_ref) +
           jnp.dot(xs, Bb_ref[...], preferred_element_type=jnp.float32))
    o32 = o32 * np.float32(_RS)
    scal = o32[:, :S + V]
    vc = o32[:, S + V:S + 2 * V]
    vecs = [vc * sh1[:, m:m + 1] for m in range(3)]
    out_ref[...] = jnp.concatenate([scal] + vecs, axis=1)


def _edge1_body(rel_ref, xg_ref, freq_ref, rw1_ref, rb1_ref, a2d_ref,
                a2dv_ref, rh32_ref, rh8_ref, g32_ref, g8_ref, Bb_ref, Bv_ref,
                out_ref):
    d, sh1 = _geom(rel_ref[:, :3])
    h = _radial_h(d, freq_ref[...], rw1_ref[...], rb1_ref[...])
    xg = xg_ref[...]
    xs = xg[:, :S]
    xv = [xg[:, S + V * m:S + V * (m + 1)] for m in range(3)]
    dot = (xv[0] * sh1[:, 0:1] + xv[1] * sh1[:, 1:2] +
           xv[2] * sh1[:, 2:3]) * np.float32(_R3)
    u = jnp.concatenate([xs, dot], axis=1)
    o32 = (_fmm(u, h, a2d_ref, rh32_ref, g32_ref) +
           jnp.dot(u, Bb_ref[...], preferred_element_type=jnp.float32))
    scal = o32[:, :S + V] * np.float32(_RSV)
    cSV = o32[:, S + V:S + 2 * V]
    crs = [
        (xv[1] * sh1[:, 2:3] - xv[2] * sh1[:, 1:2]) * np.float32(_R2),
        (xv[2] * sh1[:, 0:1] - xv[0] * sh1[:, 2:3]) * np.float32(_R2),
        (xv[0] * sh1[:, 1:2] - xv[1] * sh1[:, 0:1]) * np.float32(_R2),
    ]
    hr8 = jnp.dot(h, rh8_ref[...], preferred_element_type=jnp.float32)
    Av = a2dv_ref[...]
    g8 = g8_ref[...]
    Bv = Bv_ref[...]
    vecs = []
    for m in range(3):
        G = jnp.concatenate([xv[m], crs[m]], axis=1)
        tv = jnp.dot(G, Av, preferred_element_type=jnp.float32)
        v0c = (jnp.dot(tv * hr8, g8, preferred_element_type=jnp.float32) +
               jnp.dot(G, Bv, preferred_element_type=jnp.float32))
        vecs.append((cSV * sh1[:, m:m + 1] + v0c) * np.float32(_RS2V))
    out_ref[...] = jnp.concatenate([scal] + vecs, axis=1)


def _edge2_body(rel_ref, xg_ref, freq_ref, rw1_ref, rb1_ref, a2d_ref, rh_ref,
                g_ref, Bb_ref, out_ref):
    d, sh1 = _geom(rel_ref[:, :3])
    h = _radial_h(d, freq_ref[...], rw1_ref[...], rb1_ref[...])
    xg = xg_ref[...]
    xs = xg[:, :S]
    xv = [xg[:, S + V * m:S + V * (m + 1)] for m in range(3)]
    dot = (xv[0] * sh1[:, 0:1] + xv[1] * sh1[:, 1:2] +
           xv[2] * sh1[:, 2:3]) * np.float32(_R3)
    u = jnp.concatenate([xs, dot], axis=1)
    out_ref[...] = (_fmm(u, h, a2d_ref, rh_ref, g_ref) +
                    jnp.dot(u, Bb_ref[...],
                            preferred_element_type=jnp.float32)
                    ) * np.float32(_RSV)


def _run_edge(body, n_extra, rel, xg, consts, out_dim, eb):
    E = rel.shape[0]
    grid = E // eb
    full = lambda a: pl.BlockSpec(a.shape, lambda i: (0,) * a.ndim)
    in_specs = [
        pl.BlockSpec((eb, rel.shape[1]), lambda i: (i, 0)),
        pl.BlockSpec((eb, xg.shape[1]), lambda i: (i, 0)),
    ] + [full(c) for c in consts]
    return pl.pallas_call(
        body,
        grid=(grid,),
        in_specs=in_specs,
        out_specs=pl.BlockSpec((eb, out_dim), lambda i: (i, 0)),
        out_shape=jax.ShapeDtypeStruct((E, out_dim), jnp.float32),
    )(rel, xg, *consts)


def _embed_body(z_ref, embed_ref, out_ref):
    z = z_ref[...]
    emb = embed_ref[...]
    acc = jnp.zeros((z.shape[0], S), jnp.float32)
    for c in range(emb.shape[0]):
        acc = acc + jnp.where(z == c, 1.0, 0.0) * emb[c][None, :]
    out_ref[...] = acc


def _gate_body(ha_ref, hb_ref, out_ref):
    h = ha_ref[...] + hb_ref[...]
    scal = _silu(h[:, :S])
    g = _sigmoid(h[:, S:S + V])
    vecs = [h[:, S + V + V * m:S + V + V * (m + 1)] * g for m in range(3)]
    pad = jnp.zeros((h.shape[0], V), h.dtype)
    out_ref[...] = jnp.concatenate([scal] + vecs + [pad], axis=1)


def _readout_body(ha_ref, hb_ref, b_ref, dw1_ref, db1_ref, dw2_ref, db2_ref,
                  out_ref, acc_ref, *, ng, nblocks):
    i = pl.program_id(0)

    @pl.when(i == 0)
    def _init():
        acc_ref[...] = jnp.zeros_like(acc_ref)

    h = ha_ref[...] + hb_ref[...]
    t = jax.nn.relu(jnp.dot(h, dw1_ref[...],
                            preferred_element_type=jnp.float32) + db1_ref[...])
    y = jnp.dot(t, dw2_ref[...], preferred_element_type=jnp.float32) \
        + db2_ref[...]
    b = b_ref[...]
    gid = jax.lax.broadcasted_iota(jnp.int32, (1, ng), 1)
    onehot = jnp.where(b == gid, 1.0, 0.0)
    sums = jnp.sum(onehot * y, axis=0, keepdims=True)
    cnts = jnp.sum(onehot, axis=0, keepdims=True)
    acc_ref[0:1, :] += sums
    acc_ref[1:2, :] += cnts

    @pl.when(i == nblocks - 1)
    def _fin():
        out_ref[...] = acc_ref[0:1, :] / jnp.maximum(acc_ref[1:2, :], 1.0)


_NW = 32
_CW = 128


def _sc_mesh():
    return plsc.VectorSubcoreMesh(core_axis_name="c", subcore_axis_name="s")


def _wid():
    return lax.axis_index("s") * 2 + lax.axis_index("c")


def _geom_sc(n, ept):
    ch = ept // _CW

    def body(pos_hbm, src_hbm, tgt_hbm, rel_hbm, pos_v, si_v, ti_v, rel_v):
        w = _wid()
        pltpu.sync_copy(pos_hbm, pos_v)
        pltpu.sync_copy(src_hbm.at[w], si_v)
        pltpu.sync_copy(tgt_hbm.at[w], ti_v)
        iota = lax.broadcasted_iota(jnp.int32, (16,), 0)

        def step(i, carry):
            j = i // 8
            k = i % 8
            sl = pl.ds(k * 16, 16)
            si = si_v[j, sl] * 3
            ti = ti_v[j, sl] * 3
            row = (i * 16 + iota) * 4
            for c in range(3):
                gs = plsc.load_gather(pos_v, [si + c])
                gt = plsc.load_gather(pos_v, [ti + c])
                plsc.store_scatter(rel_v, [row + c], gt - gs)
            return carry

        lax.fori_loop(0, ch * 8, step, 0)
        pltpu.sync_copy(rel_v, rel_hbm.at[pl.ds(w * ept * 4, ept * 4)])

    return pl.kernel(
        body,
        mesh=_sc_mesh(),
        compiler_params=pltpu.CompilerParams(needs_layout_passes=False),
        out_type=jax.ShapeDtypeStruct((_NW * ept * 4,), jnp.float32),
        scratch_types=[
            pltpu.VMEM((n * 3,), jnp.float32),
            pltpu.VMEM((ch, _CW), jnp.int32),
            pltpu.VMEM((ch, _CW), jnp.int32),
            pltpu.VMEM((ept * 4,), jnp.float32),
        ],
    )


def _gath_sc(n, ept, f):
    ch = ept // _CW

    def body(tab_hbm, idx_hbm, out_hbm, idx_v, buf_v, sem):
        w = _wid()
        pltpu.sync_copy(idx_hbm.at[w], idx_v)

        def step(j, carry):
            pltpu.async_copy(tab_hbm.at[idx_v.at[j]], buf_v, sem).wait()
            pltpu.sync_copy(buf_v, out_hbm.at[pl.ds(w * ept + j * _CW, _CW)])
            return carry

        lax.fori_loop(0, ch, step, 0)

    return pl.kernel(
        body,
        mesh=_sc_mesh(),
        compiler_params=pltpu.CompilerParams(needs_layout_passes=False,
                                             use_tc_tiling_on_sc=False),
        out_type=jax.ShapeDtypeStruct((_NW * ept, f), jnp.float32),
        scratch_types=[
            pltpu.VMEM((ch, _CW), jnp.int32),
            pltpu.VMEM((_CW, f), jnp.float32),
            pltpu.SemaphoreType.DMA,
        ],
    )


def _scat_sc(n, ept, f):
    ch = ept // _CW
    nrow = n + (-n % 128) + 128
    rpt = nrow // 16

    def body(msg_hbm, tgt_hbm, zero_hbm, out_hbm, idx_v, buf_v, acc_sh):
        c = lax.axis_index("c")
        s = lax.axis_index("s")
        w = s * 2 + c
        pltpu.sync_copy(tgt_hbm.at[w], idx_v)
        pltpu.sync_copy(zero_hbm.at[pl.ds(s * rpt, rpt)],
                        acc_sh.at[pl.ds(s * rpt, rpt)])
        plsc.subcore_barrier()

        def step(j, carry):
            pltpu.sync_copy(msg_hbm.at[pl.ds(w * ept + j * _CW, _CW)], buf_v)
            for k in range(_CW // 16):
                idx = idx_v[j, pl.ds(k * 16, 16)]
                pltpu.sync_copy(buf_v.at[pl.ds(k * 16, 16)],
                                acc_sh.at[idx], add=True)
            return carry

        lax.fori_loop(0, ch, step, 0)
        plsc.subcore_barrier()
        pltpu.sync_copy(acc_sh.at[pl.ds(s * rpt, rpt)],
                        out_hbm.at[c, pl.ds(s * rpt, rpt)])

    return pl.kernel(
        body,
        mesh=_sc_mesh(),
        compiler_params=pltpu.CompilerParams(needs_layout_passes=False,
                                             use_tc_tiling_on_sc=False),
        out_type=jax.ShapeDtypeStruct((2, nrow, f), jnp.float32),
        scratch_types=[
            pltpu.VMEM((ch, _CW), jnp.int32),
            pltpu.VMEM((_CW, f), jnp.float32),
            pltpu.VMEM_SHARED((nrow, f), jnp.float32),
        ],
    )


def kernel(pos, z, edge_index, batch, embed,
           freq0, rw1_0, rb1_0, rw2_0, rb2_0,
           freq1, rw1_1, rb1_1, rw2_1, rb2_1,
           freq2, rw1_2, rb1_2, rw2_2, rb2_2,
           dw1, db1, dw2, db2):
    N = pos.shape[0]
    E = edge_index.shape[1]
    NG = 16
    MAXZ = embed.shape[0]

    src = edge_index[0].astype(jnp.int32)
    tgt = edge_index[1].astype(jnp.int32)

    ep = -E % (_NW * _CW)
    E_pad = E + ep
    ept = E_pad // _NW
    src_p = jnp.concatenate([src, jnp.zeros((ep,), jnp.int32)])
    tgt_p = jnp.concatenate([tgt, jnp.full((ep,), N, jnp.int32)])
    src3 = src_p.reshape(_NW, ept // _CW, _CW)
    tgt3 = tgt_p.reshape(_NW, ept // _CW, _CW)

    eb = 5120 if E_pad % 5120 == 0 else E_pad
    nb = 1000 if N % 1000 == 0 else N

    A0 = jnp.concatenate([
        rw2_0[:, :S * (S + V)].reshape(S, S, S + V).reshape(S * S, S + V),
        rw2_0[:, S * (S + V):].reshape(S, S, V).reshape(S * S, V)], axis=1)
    B0 = jnp.concatenate([
        rb2_0[:S * (S + V)].reshape(S, S + V),
        rb2_0[S * (S + V):].reshape(S, V)], axis=1)

    A1s = rw2_1[:, :576].reshape(S, S + V, S + V).reshape(S * (S + V), S + V)
    A1sv = jnp.zeros((S, S + V, V), jnp.float32).at[:, :S, :].set(
        rw2_1[:, 576:704].reshape(S, S, V)).reshape(S * (S + V), V)
    A1 = jnp.concatenate([A1s, A1sv], axis=1)
    B1s = rb2_1[:576].reshape(S + V, S + V)
    B1sv = jnp.zeros((S + V, V), jnp.float32).at[:S, :].set(
        rb2_1[576:704].reshape(S, V))
    B1 = jnp.concatenate([B1s, B1sv], axis=1)
    A1v = rw2_1[:, 704:832].reshape(S, 2 * V, V).reshape(S * 2 * V, V)
    B1v = rb2_1[704:832].reshape(2 * V, V)

    A2 = rw2_2.reshape(S, S + V, S).reshape(S * (S + V), S)
    B2 = rb2_2.reshape(S + V, S)

    def _a2d(a, f, o):
        return a.reshape(S, f, o).transpose(1, 0, 2).reshape(f, S * o)

    def _rh(o):
        return jnp.asarray(np.kron(np.eye(S), np.ones((1, o))), jnp.float32)

    def _gm(o):
        return jnp.asarray(np.tile(np.eye(o), (S, 1)), jnp.float32)

    A2d0 = _a2d(A0, S, 2 * S)
    A2d1 = _a2d(A1, S + V, 2 * S)
    A2dv = _a2d(A1v, 2 * V, V)
    A2d2 = _a2d(A2, S + V, S)
    RH32, RH16, RH8 = _rh(2 * S), _rh(S), _rh(V)
    G32, G16, G8 = _gm(2 * S), _gm(S), _gm(V)

    f0 = freq0.reshape(1, NB)
    f1 = freq1.reshape(1, NB)
    f2 = freq2.reshape(1, NB)
    rb1_0r = rb1_0.reshape(1, S)
    rb1_1r = rb1_1.reshape(1, S)
    rb1_2r = rb1_2.reshape(1, S)

    rel = _geom_sc(N, ept)(pos.reshape(-1), src3, tgt3).reshape(E_pad, 4)

    x0 = pl.pallas_call(
        _embed_body,
        grid=(N // nb,),
        in_specs=[pl.BlockSpec((nb, 1), lambda i: (i, 0)),
                  pl.BlockSpec((MAXZ, S), lambda i: (0, 0))],
        out_specs=pl.BlockSpec((nb, S), lambda i: (i, 0)),
        out_shape=jax.ShapeDtypeStruct((N, S), jnp.float32),
    )(z.astype(jnp.int32).reshape(N, 1), embed)

    nrow = N + (-N % 128) + 128
    zeros48 = jnp.zeros((nrow, 48), jnp.float32)
    zeros16 = jnp.zeros((nrow, 16), jnp.float32)

    def _gate(hp):
        return pl.pallas_call(
            _gate_body,
            grid=(N // nb,),
            in_specs=[pl.BlockSpec((nb, 48), lambda i: (i, 0)),
                      pl.BlockSpec((nb, 48), lambda i: (i, 0))],
            out_specs=pl.BlockSpec((nb, 48), lambda i: (i, 0)),
            out_shape=jax.ShapeDtypeStruct((N, 48), jnp.float32),
        )(hp[0, :N], hp[1, :N])

    xg = _gath_sc(N, ept, 16)(x0, src3)
    msg = _run_edge(_edge0_body, 5, rel, xg,
                    [f0, rw1_0, rb1_0r, A2d0, RH32, G32, B0], 48, eb)
    x = _gate(_scat_sc(N, ept, 48)(msg, tgt3, zeros48))

    xg = _gath_sc(N, ept, 48)(x, src3)
    msg = _run_edge(_edge1_body, 7, rel, xg,
                    [f1, rw1_1, rb1_1r, A2d1, A2dv, RH32, RH8, G32, G8,
                     B1, B1v], 48, eb)
    x = _gate(_scat_sc(N, ept, 48)(msg, tgt3, zeros48))

    xg = _gath_sc(N, ept, 48)(x, src3)
    msg = _run_edge(_edge2_body, 5, rel, xg,
                    [f2, rw1_2, rb1_2r, A2d2, RH16, G16, B2], 16, eb)
    hp = _scat_sc(N, ept, 16)(msg, tgt3, zeros16)

    nblocks = N // nb
    out = pl.pallas_call(
        functools.partial(_readout_body, ng=NG, nblocks=nblocks),
        grid=(nblocks,),
        in_specs=[pl.BlockSpec((nb, S), lambda i: (i, 0)),
                  pl.BlockSpec((nb, S), lambda i: (i, 0)),
                  pl.BlockSpec((nb, 1), lambda i: (i, 0)),
                  pl.BlockSpec((S, 2 * S), lambda i: (0, 0)),
                  pl.BlockSpec((1, 2 * S), lambda i: (0, 0)),
                  pl.BlockSpec((2 * S, 1), lambda i: (0, 0)),
                  pl.BlockSpec((1, 1), lambda i: (0, 0))],
        out_specs=pl.BlockSpec((1, NG), lambda i: (0, 0)),
        out_shape=jax.ShapeDtypeStruct((1, NG), jnp.float32),
        scratch_shapes=[pltpu.VMEM((2, NG), jnp.float32)],
    )(hp[0, :N], hp[1, :N], batch.astype(jnp.int32).reshape(N, 1), dw1,
      db1.reshape(1, 2 * S), dw2, db2.reshape(1, 1))
    return out[0]


def _gather_rows(table, idx):
    return table[idx]


def _segsum(vals, idx, n):
    return jax.ops.segment_sum(vals, idx, num_segments=n + 1)[:n]


# speedup vs baseline: 2.3258x; 1.0005x over previous
import functools
import numpy as np
import jax
import jax.numpy as jnp
from jax import lax
from jax.experimental import pallas as pl
from jax.experimental.pallas import tpu as pltpu
from jax.experimental.pallas import tpu_sc as plsc

S = 16
V = 8
NB = 10
CUT = 4.0

_RS = 1.0 / np.sqrt(S)
_R3 = 1.0 / np.sqrt(3.0)
_RSV = 1.0 / np.sqrt(S + V)
_RS2V = 1.0 / np.sqrt(S + 2 * V)
_R2 = 1.0 / np.sqrt(2.0)


def _sigmoid(x):
    return 1.0 / (1.0 + jnp.exp(-x))


def _silu(x):
    return x * _sigmoid(x)


def _geom(rel):
    d = jnp.sqrt(jnp.sum(rel * rel, axis=1, keepdims=True))
    dn = jnp.maximum(d, 1e-9)
    sh1 = np.float32(np.sqrt(3.0)) * rel / dn
    return d, sh1


def _radial_h(d, freq, rw1, rb1):
    x = jnp.maximum(d * np.float32(1.0 / CUT), 1e-6)
    xp = x ** 5
    env = 1.0 / x + (-28.0) * xp + 48.0 * xp * x + (-21.0) * xp * x * x
    env = jnp.where(x < 1.0, env, 0.0)
    rbf = jnp.sin(freq * x) * env
    return _silu(jnp.dot(rbf, rw1, preferred_element_type=jnp.float32) + rb1)


def _fmm(x, h, a2d_ref, rh_ref, g_ref):
    t = jnp.dot(x, a2d_ref[...], preferred_element_type=jnp.float32)
    hr = jnp.dot(h, rh_ref[...], preferred_element_type=jnp.float32)
    return jnp.dot(t * hr, g_ref[...], preferred_element_type=jnp.float32)


def _edge0_body(rel_ref, xg_ref, freq_ref, rw1_ref, rb1_ref, a2d_ref, rh_ref,
                g_ref, Bb_ref, out_ref):
    d, sh1 = _geom(rel_ref[:, :3])
    h = _radial_h(d, freq_ref[...], rw1_ref[...], rb1_ref[...])
    xs = xg_ref[...]
    o32 = (_fmm(xs, h, a2d_ref, rh_ref, g_ref) +
           jnp.dot(xs, Bb_ref[...], preferred_element_type=jnp.float32))
    o32 = o32 * np.float32(_RS)
    scal = o32[:, :S + V]
    vc = o32[:, S + V:S + 2 * V]
    vecs = [vc * sh1[:, m:m + 1] for m in range(3)]
    out_ref[...] = jnp.concatenate([scal] + vecs, axis=1)


def _edge1_body(rel_ref, xg_ref, freq_ref, rw1_ref, rb1_ref, a2d_ref,
                a2dv_ref, rh32_ref, rh8_ref, g32_ref, g8_ref, Bb_ref, Bv_ref,
                out_ref):
    d, sh1 = _geom(rel_ref[:, :3])
    h = _radial_h(d, freq_ref[...], rw1_ref[...], rb1_ref[...])
    xg = xg_ref[...]
    xs = xg[:, :S]
    xv = [xg[:, S + V * m:S + V * (m + 1)] for m in range(3)]
    dot = (xv[0] * sh1[:, 0:1] + xv[1] * sh1[:, 1:2] +
           xv[2] * sh1[:, 2:3]) * np.float32(_R3)
    u = jnp.concatenate([xs, dot], axis=1)
    o32 = (_fmm(u, h, a2d_ref, rh32_ref, g32_ref) +
           jnp.dot(u, Bb_ref[...], preferred_element_type=jnp.float32))
    scal = o32[:, :S + V] * np.float32(_RSV)
    cSV = o32[:, S + V:S + 2 * V]
    crs = [
        (xv[1] * sh1[:, 2:3] - xv[2] * sh1[:, 1:2]) * np.float32(_R2),
        (xv[2] * sh1[:, 0:1] - xv[0] * sh1[:, 2:3]) * np.float32(_R2),
        (xv[0] * sh1[:, 1:2] - xv[1] * sh1[:, 0:1]) * np.float32(_R2),
    ]
    hr8 = jnp.dot(h, rh8_ref[...], preferred_element_type=jnp.float32)
    Av = a2dv_ref[...]
    g8 = g8_ref[...]
    Bv = Bv_ref[...]
    vecs = []
    for m in range(3):
        G = jnp.concatenate([xv[m], crs[m]], axis=1)
        tv = jnp.dot(G, Av, preferred_element_type=jnp.float32)
        v0c = (jnp.dot(tv * hr8, g8, preferred_element_type=jnp.float32) +
               jnp.dot(G, Bv, preferred_element_type=jnp.float32))
        vecs.append((cSV * sh1[:, m:m + 1] + v0c) * np.float32(_RS2V))
    out_ref[...] = jnp.concatenate([scal] + vecs, axis=1)


def _edge2_body(rel_ref, xg_ref, freq_ref, rw1_ref, rb1_ref, a2d_ref, rh_ref,
                g_ref, Bb_ref, out_ref):
    d, sh1 = _geom(rel_ref[:, :3])
    h = _radial_h(d, freq_ref[...], rw1_ref[...], rb1_ref[...])
    xg = xg_ref[...]
    xs = xg[:, :S]
    xv = [xg[:, S + V * m:S + V * (m + 1)] for m in range(3)]
    dot = (xv[0] * sh1[:, 0:1] + xv[1] * sh1[:, 1:2] +
           xv[2] * sh1[:, 2:3]) * np.float32(_R3)
    u = jnp.concatenate([xs, dot], axis=1)
    out_ref[...] = (_fmm(u, h, a2d_ref, rh_ref, g_ref) +
                    jnp.dot(u, Bb_ref[...],
                            preferred_element_type=jnp.float32)
                    ) * np.float32(_RSV)


def _run_edge(body, rel, xg, consts, out_dim, eb):
    E = rel.shape[0]
    grid = E // eb
    full = lambda a: pl.BlockSpec(a.shape, lambda i: (0,) * a.ndim)
    in_specs = [
        pl.BlockSpec((eb, rel.shape[1]), lambda i: (i, 0)),
        pl.BlockSpec((eb, xg.shape[1]), lambda i: (i, 0)),
    ] + [full(c) for c in consts]
    return pl.pallas_call(
        body,
        grid=(grid,),
        in_specs=in_specs,
        out_specs=pl.BlockSpec((eb, out_dim), lambda i: (i, 0)),
        out_shape=jax.ShapeDtypeStruct((E, out_dim), jnp.float32),
    )(rel, xg, *consts)


def _embed_body(z_ref, embed_ref, out_ref):
    z = z_ref[...]
    emb = embed_ref[...]
    acc = jnp.zeros((z.shape[0], S), jnp.float32)
    for c in range(emb.shape[0]):
        acc = acc + jnp.where(z == c, 1.0, 0.0) * emb[c][None, :]
    out_ref[...] = acc


def _gate_body(ha_ref, hb_ref, out_ref):
    h = ha_ref[...] + hb_ref[...]
    scal = _silu(h[:, :S])
    g = _sigmoid(h[:, S:S + V])
    vecs = [h[:, S + V + V * m:S + V + V * (m + 1)] * g for m in range(3)]
    pad = jnp.zeros((h.shape[0], V), h.dtype)
    out_ref[...] = jnp.concatenate([scal] + vecs + [pad], axis=1)


def _readout_body(ha_ref, hb_ref, b_ref, dw1_ref, db1_ref, dw2_ref, db2_ref,
                  out_ref, acc_ref, *, ng, nblocks):
    i = pl.program_id(0)

    @pl.when(i == 0)
    def _init():
        acc_ref[...] = jnp.zeros_like(acc_ref)

    h = ha_ref[...] + hb_ref[...]
    t = jax.nn.relu(jnp.dot(h, dw1_ref[...],
                            preferred_element_type=jnp.float32) + db1_ref[...])
    y = jnp.dot(t, dw2_ref[...], preferred_element_type=jnp.float32) \
        + db2_ref[...]
    b = b_ref[...]
    gid = jax.lax.broadcasted_iota(jnp.int32, (1, ng), 1)
    onehot = jnp.where(b == gid, 1.0, 0.0)
    sums = jnp.sum(onehot * y, axis=0, keepdims=True)
    cnts = jnp.sum(onehot, axis=0, keepdims=True)
    acc_ref[0:1, :] += sums
    acc_ref[1:2, :] += cnts

    @pl.when(i == nblocks - 1)
    def _fin():
        out_ref[...] = acc_ref[0:1, :] / jnp.maximum(acc_ref[1:2, :], 1.0)


_NW = 32
_CW = 128


def _sc_mesh():
    return plsc.VectorSubcoreMesh(core_axis_name="c", subcore_axis_name="s")


def _wid():
    return lax.axis_index("s") * 2 + lax.axis_index("c")


def _geom_sc(n, ept):
    ch = ept // _CW

    def body(pos_hbm, src_hbm, tgt_hbm, rel_hbm, pos_v, si_v, ti_v, rel_v):
        w = _wid()
        pltpu.sync_copy(pos_hbm, pos_v)
        pltpu.sync_copy(src_hbm.at[w], si_v)
        pltpu.sync_copy(tgt_hbm.at[w], ti_v)
        iota = lax.broadcasted_iota(jnp.int32, (16,), 0)

        def step(i, carry):
            j = i // 8
            k = i % 8
            sl = pl.ds(k * 16, 16)
            si = si_v[j, sl] * 3
            ti = ti_v[j, sl] * 3
            row = (i * 16 + iota) * 4
            for c in range(3):
                gs = plsc.load_gather(pos_v, [si + c])
                gt = plsc.load_gather(pos_v, [ti + c])
                plsc.store_scatter(rel_v, [row + c], gt - gs)
            return carry

        lax.fori_loop(0, ch * 8, step, 0)
        pltpu.sync_copy(rel_v, rel_hbm.at[pl.ds(w * ept * 4, ept * 4)])

    return pl.kernel(
        body,
        mesh=_sc_mesh(),
        compiler_params=pltpu.CompilerParams(needs_layout_passes=False),
        out_type=jax.ShapeDtypeStruct((_NW * ept * 4,), jnp.float32),
        scratch_types=[
            pltpu.VMEM((n * 3,), jnp.float32),
            pltpu.VMEM((ch, _CW), jnp.int32),
            pltpu.VMEM((ch, _CW), jnp.int32),
            pltpu.VMEM((ept * 4,), jnp.float32),
        ],
    )


def _gath_sc(n, ept, f):
    ch = ept // _CW

    def body(tab_hbm, idx_hbm, out_hbm, idx_v, buf_v, sem):
        w = _wid()
        pltpu.sync_copy(idx_hbm.at[w], idx_v)

        def step(j, carry):
            pltpu.async_copy(tab_hbm.at[idx_v.at[j]], buf_v, sem).wait()
            pltpu.sync_copy(buf_v, out_hbm.at[pl.ds(w * ept + j * _CW, _CW)])
            return carry

        lax.fori_loop(0, ch, step, 0)

    return pl.kernel(
        body,
        mesh=_sc_mesh(),
        compiler_params=pltpu.CompilerParams(needs_layout_passes=False,
                                             use_tc_tiling_on_sc=False),
        out_type=jax.ShapeDtypeStruct((_NW * ept, f), jnp.float32),
        scratch_types=[
            pltpu.VMEM((ch, _CW), jnp.int32),
            pltpu.VMEM((_CW, f), jnp.float32),
            pltpu.SemaphoreType.DMA,
        ],
    )


def _scat_sc(n, ept, f):
    ch = ept // _CW
    nrow = n + (-n % 128) + 128
    rpt = nrow // 16

    def body(msg_hbm, tgt_hbm, zero_hbm, out_hbm, idx_v, buf_v, acc_sh):
        c = lax.axis_index("c")
        s = lax.axis_index("s")
        w = s * 2 + c
        pltpu.sync_copy(tgt_hbm.at[w], idx_v)
        pltpu.sync_copy(zero_hbm.at[pl.ds(s * rpt, rpt)],
                        acc_sh.at[pl.ds(s * rpt, rpt)])
        plsc.subcore_barrier()

        def step(j, carry):
            pltpu.sync_copy(msg_hbm.at[pl.ds(w * ept + j * _CW, _CW)], buf_v)
            for k in range(_CW // 16):
                idx = idx_v[j, pl.ds(k * 16, 16)]
                pltpu.sync_copy(buf_v.at[pl.ds(k * 16, 16)],
                                acc_sh.at[idx], add=True)
            return carry

        lax.fori_loop(0, ch, step, 0)
        plsc.subcore_barrier()
        pltpu.sync_copy(acc_sh.at[pl.ds(s * rpt, rpt)],
                        out_hbm.at[c, pl.ds(s * rpt, rpt)])

    return pl.kernel(
        body,
        mesh=_sc_mesh(),
        compiler_params=pltpu.CompilerParams(needs_layout_passes=False,
                                             use_tc_tiling_on_sc=False),
        out_type=jax.ShapeDtypeStruct((2, nrow, f), jnp.float32),
        scratch_types=[
            pltpu.VMEM((ch, _CW), jnp.int32),
            pltpu.VMEM((_CW, f), jnp.float32),
            pltpu.VMEM_SHARED((nrow, f), jnp.float32),
        ],
    )


def kernel(pos, z, edge_index, batch, embed,
           freq0, rw1_0, rb1_0, rw2_0, rb2_0,
           freq1, rw1_1, rb1_1, rw2_1, rb2_1,
           freq2, rw1_2, rb1_2, rw2_2, rb2_2,
           dw1, db1, dw2, db2):
    N = pos.shape[0]
    E = edge_index.shape[1]
    NG = 16
    MAXZ = embed.shape[0]

    src = edge_index[0].astype(jnp.int32)
    tgt = edge_index[1].astype(jnp.int32)

    ep = -E % (_NW * _CW)
    E_pad = E + ep
    ept = E_pad // _NW
    src_p = jnp.concatenate([src, jnp.zeros((ep,), jnp.int32)])
    tgt_p = jnp.concatenate([tgt, jnp.full((ep,), N, jnp.int32)])
    src3 = src_p.reshape(_NW, ept // _CW, _CW)
    tgt3 = tgt_p.reshape(_NW, ept // _CW, _CW)

    eb = 5120 if E_pad % 5120 == 0 else E_pad
    nb = 1000 if N % 1000 == 0 else N

    A0 = jnp.concatenate([
        rw2_0[:, :S * (S + V)].reshape(S, S, S + V).reshape(S * S, S + V),
        rw2_0[:, S * (S + V):].reshape(S, S, V).reshape(S * S, V)], axis=1)
    B0 = jnp.concatenate([
        rb2_0[:S * (S + V)].reshape(S, S + V),
        rb2_0[S * (S + V):].reshape(S, V)], axis=1)

    A1s = rw2_1[:, :576].reshape(S, S + V, S + V).reshape(S * (S + V), S + V)
    A1sv = jnp.zeros((S, S + V, V), jnp.float32).at[:, :S, :].set(
        rw2_1[:, 576:704].reshape(S, S, V)).reshape(S * (S + V), V)
    A1 = jnp.concatenate([A1s, A1sv], axis=1)
    B1s = rb2_1[:576].reshape(S + V, S + V)
    B1sv = jnp.zeros((S + V, V), jnp.float32).at[:S, :].set(
        rb2_1[576:704].reshape(S, V))
    B1 = jnp.concatenate([B1s, B1sv], axis=1)
    A1v = rw2_1[:, 704:832].reshape(S, 2 * V, V).reshape(S * 2 * V, V)
    B1v = rb2_1[704:832].reshape(2 * V, V)

    A2 = rw2_2.reshape(S, S + V, S).reshape(S * (S + V), S)
    B2 = rb2_2.reshape(S + V, S)

    def _a2d(a, f, o):
        return a.reshape(S, f, o).transpose(1, 0, 2).reshape(f, S * o)

    def _rh(o):
        return jnp.asarray(np.kron(np.eye(S), np.ones((1, o))), jnp.float32)

    def _gm(o):
        return jnp.asarray(np.tile(np.eye(o), (S, 1)), jnp.float32)

    A2d0 = _a2d(A0, S, 2 * S)
    A2d1 = _a2d(A1, S + V, 2 * S)
    A2dv = _a2d(A1v, 2 * V, V)
    A2d2 = _a2d(A2, S + V, S)
    RH32, RH16, RH8 = _rh(2 * S), _rh(S), _rh(V)
    G32, G16, G8 = _gm(2 * S), _gm(S), _gm(V)

    f0 = freq0.reshape(1, NB)
    f1 = freq1.reshape(1, NB)
    f2 = freq2.reshape(1, NB)
    rb1_0r = rb1_0.reshape(1, S)
    rb1_1r = rb1_1.reshape(1, S)
    rb1_2r = rb1_2.reshape(1, S)

    rel = _geom_sc(N, ept)(pos.reshape(-1), src3, tgt3).reshape(E_pad, 4)

    x0 = pl.pallas_call(
        _embed_body,
        grid=(N // nb,),
        in_specs=[pl.BlockSpec((nb, 1), lambda i: (i, 0)),
                  pl.BlockSpec((MAXZ, S), lambda i: (0, 0))],
        out_specs=pl.BlockSpec((nb, S), lambda i: (i, 0)),
        out_shape=jax.ShapeDtypeStruct((N, S), jnp.float32),
    )(z.astype(jnp.int32).reshape(N, 1), embed)

    nrow = N + (-N % 128) + 128
    zeros48 = jnp.zeros((nrow, 48), jnp.float32)
    zeros16 = jnp.zeros((nrow, 16), jnp.float32)

    def _gate(hp):
        return pl.pallas_call(
            _gate_body,
            grid=(N // nb,),
            in_specs=[pl.BlockSpec((nb, 48), lambda i: (i, 0)),
                      pl.BlockSpec((nb, 48), lambda i: (i, 0))],
            out_specs=pl.BlockSpec((nb, 48), lambda i: (i, 0)),
            out_shape=jax.ShapeDtypeStruct((N, 48), jnp.float32),
        )(hp[0, :N], hp[1, :N])

    xg = _gath_sc(N, ept, 16)(x0, src3)
    msg = _run_edge(_edge0_body, rel, xg,
                    [f0, rw1_0, rb1_0r, A2d0, RH32, G32, B0], 48, eb)
    x = _gate(_scat_sc(N, ept, 48)(msg, tgt3, zeros48))

    xg = _gath_sc(N, ept, 48)(x, src3)
    msg = _run_edge(_edge1_body, rel, xg,
                    [f1, rw1_1, rb1_1r, A2d1, A2dv, RH32, RH8, G32, G8,
                     B1, B1v], 48, eb)
    x = _gate(_scat_sc(N, ept, 48)(msg, tgt3, zeros48))

    xg = _gath_sc(N, ept, 48)(x, src3)
    msg = _run_edge(_edge2_body, rel, xg,
                    [f2, rw1_2, rb1_2r, A2d2, RH16, G16, B2], 16, eb)
    hp = _scat_sc(N, ept, 16)(msg, tgt3, zeros16)

    nblocks = N // nb
    out = pl.pallas_call(
        functools.partial(_readout_body, ng=NG, nblocks=nblocks),
        grid=(nblocks,),
        in_specs=[pl.BlockSpec((nb, S), lambda i: (i, 0)),
                  pl.BlockSpec((nb, S), lambda i: (i, 0)),
                  pl.BlockSpec((nb, 1), lambda i: (i, 0)),
                  pl.BlockSpec((S, 2 * S), lambda i: (0, 0)),
                  pl.BlockSpec((1, 2 * S), lambda i: (0, 0)),
                  pl.BlockSpec((2 * S, 1), lambda i: (0, 0)),
                  pl.BlockSpec((1, 1), lambda i: (0, 0))],
        out_specs=pl.BlockSpec((1, NG), lambda i: (0, 0)),
        out_shape=jax.ShapeDtypeStruct((1, NG), jnp.float32),
        scratch_shapes=[pltpu.VMEM((2, NG), jnp.float32)],
    )(hp[0, :N], hp[1, :N], batch.astype(jnp.int32).reshape(N, 1), dw1,
      db1.reshape(1, 2 * S), dw2, db2.reshape(1, 1))
    return out[0]
